# baseline (speedup 1.0000x reference)
import functools

import jax
import jax.numpy as jnp
from jax import lax
from jax.experimental import pallas as pl
from jax.experimental.pallas import tpu as pltpu
from jax.experimental.pallas import tpu_sc as plsc

_N = 10000
_E = 320000
_D = 128
_NC = 2
_NS = 16
_NW = _NC * _NS
_EPW = _E // _NW
_CH = 80
_NCH = _EPW // _CH
_RB = 1000
_EB = 4000
_CW = 8
_NP = 10240

_f32 = jnp.float32
_bf16 = jnp.bfloat16
_i32 = jnp.int32


def _ln_tables_body(h_ref, g_ref, b_ref, wa_ref, wb_ref, bb_ref,
                    hn_ref, pa_ref, pb_ref):
    hb = h_ref[...]
    mu = jnp.mean(hb, axis=1, keepdims=True)
    ctr = hb - mu
    var = jnp.mean(ctr * ctr, axis=1, keepdims=True)
    hn = ctr * lax.rsqrt(var + 1e-5) * g_ref[...] + b_ref[...]
    hn_ref[...] = hn
    pa_ref[...] = jnp.dot(hn, wa_ref[...], preferred_element_type=_f32)
    pb_ref[...] = jnp.dot(hn, wb_ref[...], preferred_element_type=_f32) + bb_ref[...]


def _ln_tables(h, g1, bt1, w1a, w1b, be1):
    full = lambda shp: pl.BlockSpec(shp, lambda i: (0,) * len(shp))
    return pl.pallas_call(
        _ln_tables_body,
        grid=(_N // _RB,),
        in_specs=[
            pl.BlockSpec((_RB, _D), lambda i: (i, 0)),
            full((1, _D)), full((1, _D)),
            full((_D, _D)), full((_D, _D)), full((1, _D)),
        ],
        out_specs=[pl.BlockSpec((_RB, _D), lambda i: (i, 0))] * 3,
        out_shape=[jax.ShapeDtypeStruct((_N, _D), _f32)] * 3,
    )(h, g1, bt1, w1a, w1b, be1)


def _gather_sc(pa, pb, x0, x1, x2, row, col):
    mesh = plsc.VectorSubcoreMesh(core_axis_name="c", subcore_axis_name="s",
                                  num_cores=_NC, num_subcores=_NS)

    @functools.partial(
        pl.kernel,
        out_type=(jax.ShapeDtypeStruct((_E, _D), _f32),
                  jax.ShapeDtypeStruct((_E, _CW), _f32)),
        mesh=mesh,
        compiler_params=pltpu.CompilerParams(needs_layout_passes=False),
        scratch_types=[
            pltpu.VMEM((2, _CH), _i32),
            pltpu.VMEM((2, _CH), _i32),
            pltpu.VMEM((_CH, _D), _f32),
            pltpu.VMEM((_CH, _D), _f32),
            pltpu.VMEM((_CH, _D), _f32),
            pltpu.VMEM((_CH, _D), _f32),
            pltpu.VMEM((2, _CH, _CW), _f32),
            pltpu.VMEM((_N,), _f32),
            pltpu.VMEM((_N,), _f32),
            pltpu.VMEM((_N,), _f32),
            pltpu.SemaphoreType.DMA,
            pltpu.SemaphoreType.DMA,
            pltpu.SemaphoreType.DMA,
        ],
    )
    def k(pa_h, pb_h, x0_h, x1_h, x2_h, row_h, col_h,
          sa_h, d2_h,
          idxr2, idxc2, bufa0, bufa1, bufb0, bufb1, d2b2, x0v, x1v, x2v,
          semg0, semg1, semw):
        semg = (semg0, semg1)
        c = lax.axis_index("c")
        s = lax.axis_index("s")
        ebase = (c * _NS + s) * _EPW
        pltpu.sync_copy(x0_h, x0v)
        pltpu.sync_copy(x1_h, x1v)
        pltpu.sync_copy(x2_h, x2v)
        lanes = lax.iota(_i32, 16)
        zeros16 = jnp.zeros((16,), _i32)
        bufa = (bufa0, bufa1)
        bufb = (bufb0, bufb1)

        def load_idx(cc, S):
            base = ebase + cc * _CH
            pltpu.sync_copy(row_h.at[pl.ds(base, _CH)], idxr2.at[S])
            pltpu.sync_copy(col_h.at[pl.ds(base, _CH)], idxc2.at[S])

        def start_gathers(S):
            pltpu.async_copy(pa_h.at[idxr2.at[S]], bufa[S], semg[S])
            pltpu.async_copy(pb_h.at[idxc2.at[S]], bufb[S], semg[S])

        def drain_gathers(S):
            pltpu.make_async_copy(pa_h.at[idxr2.at[S]], bufa[S], semg[S]).wait()
            pltpu.make_async_copy(pb_h.at[idxc2.at[S]], bufb[S], semg[S]).wait()

        def start_wb(cc, S):
            base = ebase + cc * _CH
            pltpu.async_copy(bufa[S], sa_h.at[pl.ds(base, _CH)], semw)
            pltpu.async_copy(d2b2.at[S], d2_h.at[pl.ds(base, _CH)], semw)

        def drain_wb(cc, S):
            base = ebase + cc * _CH
            pltpu.make_async_copy(bufa[S], sa_h.at[pl.ds(base, _CH)], semw).wait()
            pltpu.make_async_copy(d2b2.at[S], d2_h.at[pl.ds(base, _CH)], semw).wait()

        def add_rows(S):
            def row_add(r, carry2):
                for l in range(_D // 16):
                    sl = pl.ds(l * 16, 16)
                    bufa[S][r, sl] = bufa[S][r, sl] + bufb[S][r, sl]
                return carry2

            lax.fori_loop(0, _CH, row_add, 0)

        def compute_d2(S):
            def dist_group(g, carry2):
                ir = idxr2[S, pl.ds(g * 16, 16)]
                ic = idxc2[S, pl.ds(g * 16, 16)]
                dx = plsc.load_gather(x0v, [ir]) - plsc.load_gather(x0v, [ic])
                dy = plsc.load_gather(x1v, [ir]) - plsc.load_gather(x1v, [ic])
                dz = plsc.load_gather(x2v, [ir]) - plsc.load_gather(x2v, [ic])
                d2v = dx * dx + dy * dy + dz * dz
                plsc.store_scatter(d2b2.at[S], [g * 16 + lanes, zeros16], d2v)
                return carry2

            lax.fori_loop(0, _CH // 16, dist_group, 0)

        def phase(cc, S, Sp):
            @pl.when(cc < _NCH)
            def _():
                @pl.when(cc > 0)
                def _():
                    drain_wb(cc - 1, Sp)

                @pl.when(cc + 1 < _NCH)
                def _():
                    load_idx(cc + 1, Sp)
                    start_gathers(Sp)

                drain_gathers(S)
                compute_d2(S)
                add_rows(S)
                start_wb(cc, S)

        load_idx(0, 0)
        start_gathers(0)

        def body(j, carry):
            phase(2 * j, 0, 1)
            phase(2 * j + 1, 1, 0)
            return carry

        lax.fori_loop(0, (_NCH + 2) // 2, body, 0)
        drain_wb(_NCH - 1, (_NCH - 1) % 2)

    return k(pa, pb, x0, x1, x2, row, col)


def _edge_mlp_body(sa_ref, d2_ref, wd_ref, w2_ref, b2_ref, m_ref):
    dist = jnp.sqrt(d2_ref[...][:, 0:1])
    z = (sa_ref[...] + dist * wd_ref[...]).astype(_bf16)
    m1 = z * jax.nn.sigmoid(z)
    z2 = ((jnp.dot(m1, w2_ref[...], preferred_element_type=_f32)
           + b2_ref[...]).astype(_bf16))
    m_ref[...] = (z2 * jax.nn.sigmoid(z2)).astype(_f32)


def _edge_mlp(sa, d2, wd, w2, b2):
    full = lambda shp: pl.BlockSpec(shp, lambda i: (0,) * len(shp))
    return pl.pallas_call(
        _edge_mlp_body,
        grid=(_E // _EB,),
        in_specs=[
            pl.BlockSpec((_EB, _D), lambda i: (i, 0)),
            pl.BlockSpec((_EB, _CW), lambda i: (i, 0)),
            full((1, _D)), full((_D, _D)), full((1, _D)),
        ],
        out_specs=pl.BlockSpec((_EB, _D), lambda i: (i, 0)),
        out_shape=jax.ShapeDtypeStruct((_E, _D), _f32),
    )(sa, d2, wd, w2.astype(_bf16), b2)


def _scatter_sc(m, col, z128, zcnt):
    mesh = plsc.VectorSubcoreMesh(core_axis_name="c", subcore_axis_name="s",
                                  num_cores=_NC, num_subcores=_NS)

    @functools.partial(
        pl.kernel,
        out_type=(jax.ShapeDtypeStruct((_NC, _N, _D), _f32),
                  jax.ShapeDtypeStruct((_NW * _NP,), _f32)),
        mesh=mesh,
        compiler_params=pltpu.CompilerParams(needs_layout_passes=False),
        scratch_types=[
            pltpu.VMEM((2, _CH), _i32),
            pltpu.VMEM((_CH, _D), _f32),
            pltpu.VMEM((_CH, _D), _f32),
            pltpu.VMEM((_NP + 16,), _f32),
            pltpu.VMEM_SHARED((_N, _D), _f32),
            pltpu.SemaphoreType.DMA,
            pltpu.SemaphoreType.DMA,
            pltpu.SemaphoreType.DMA,
        ],
    )
    def k(m_h, col_h, z128_h, zcnt_h, msum_h, cnt_h, idx2, data0, data1,
          cntv, msh, seml, semsc0, semsc1):
        semsc = (semsc0, semsc1)
        data = (data0, data1)
        c = lax.axis_index("c")
        s = lax.axis_index("s")
        wid = c * _NS + s
        ebase = wid * _EPW
        pltpu.sync_copy(zcnt_h, cntv)
        lanes = lax.iota(_i32, 16)

        def count_chunk(S):
            def count16(g, carry2):
                ivvec = idx2[S, pl.ds(g * 16, 16)]
                for jj in range(16):
                    iv = ivvec[jj]
                    cbase = lax.shift_left(lax.shift_right_logical(iv, 3), 3)
                    lane = iv - cbase
                    cntv[pl.ds(cbase, 16)] = (cntv[pl.ds(cbase, 16)]
                                              + (lanes == lane).astype(_f32))
                return carry2

            lax.fori_loop(0, _CH // 16, count16, 0)

        def start_loads(cc, S):
            base = ebase + cc * _CH
            pltpu.async_copy(col_h.at[pl.ds(base, _CH)], idx2.at[S], seml)
            pltpu.async_copy(m_h.at[pl.ds(base, _CH)], data[S], seml)

        def drain_loads(cc, S):
            base = ebase + cc * _CH
            pltpu.make_async_copy(col_h.at[pl.ds(base, _CH)], idx2.at[S], seml).wait()
            pltpu.make_async_copy(m_h.at[pl.ds(base, _CH)], data[S], seml).wait()

        def start_scatter(S):
            pltpu.async_copy(data[S], msh.at[idx2.at[S]], semsc[S], add=True)

        def drain_scatter(S):
            pltpu.make_async_copy(data[S], msh.at[idx2.at[S]], semsc[S]).wait()

        @pl.when(s == 0)
        def _():
            pltpu.sync_copy(z128_h, msh)

        plsc.subcore_barrier()
        start_loads(0, 0)

        def phase(cc, S, Sp):
            @pl.when(cc < _NCH)
            def _():
                drain_loads(cc, S)

                @pl.when(cc > 0)
                def _():
                    drain_scatter(Sp)

                @pl.when(cc + 1 < _NCH)
                def _():
                    start_loads(cc + 1, Sp)

                start_scatter(S)
                count_chunk(S)

        def body(j, carry):
            phase(2 * j, 0, 1)
            phase(2 * j + 1, 1, 0)
            return carry

        lax.fori_loop(0, (_NCH + 2) // 2, body, 0)
        drain_scatter((_NCH - 1) % 2)
        pltpu.sync_copy(cntv.at[pl.ds(0, _NP)], cnt_h.at[pl.ds(wid * _NP, _NP)])
        plsc.subcore_barrier()
        rpt0 = 624

        @pl.when(s < _NS - 1)
        def _():
            rb = s * rpt0
            pltpu.sync_copy(msh.at[pl.ds(rb, rpt0)], msum_h.at[c, pl.ds(rb, rpt0)])

        @pl.when(s == _NS - 1)
        def _():
            rb = (_NS - 1) * rpt0
            rpt1 = _N - rb
            pltpu.sync_copy(msh.at[pl.ds(rb, rpt1)], msum_h.at[c, pl.ds(rb, rpt1)])

    return k(m, col, z128, zcnt)


def _node_body(h_ref, hn_ref, ms_ref, ct_ref, wna_ref, wnb_ref, bn1_ref,
               wn2_ref, bn2_ref, wm1_ref, bm1_ref, wm2_ref, bm2_ref,
               g2_ref, b2t_ref, out_ref):
    ms = ms_ref[0] + ms_ref[1]
    cnt = jnp.sum(ct_ref[...], axis=0)[:, 0:1]
    maggr = ms / jnp.maximum(cnt, 1.0)
    hn = hn_ref[...]
    bf = lambda v: v.astype(_bf16)
    z = (jnp.dot(bf(hn), bf(wna_ref[...]), preferred_element_type=_f32)
         + jnp.dot(bf(maggr), bf(wnb_ref[...]), preferred_element_type=_f32)
         + bn1_ref[...])
    a = z * jax.nn.sigmoid(z)
    h_delta = (jnp.dot(bf(a), bf(wn2_ref[...]), preferred_element_type=_f32)
               + bn2_ref[...])
    h1 = h_ref[...] + hn + h_delta
    mu = jnp.mean(h1, axis=1, keepdims=True)
    ctr = h1 - mu
    var = jnp.mean(ctr * ctr, axis=1, keepdims=True)
    hn2 = ctr * lax.rsqrt(var + 1e-5) * g2_ref[...] + b2t_ref[...]
    z2 = (jnp.dot(bf(hn2), bf(wm1_ref[...]), preferred_element_type=_f32)
          + bm1_ref[...])
    a2 = z2 * jax.nn.sigmoid(z2)
    out_ref[...] = (h1 + jnp.dot(bf(a2), bf(wm2_ref[...]), preferred_element_type=_f32)
                    + bm2_ref[...])


def _node_mlp(h, hn, msum2, cnt2, wna, wnb, bn1, wn2, bn2, wm1, bm1, wm2, bm2, g2, bt2):
    full = lambda shp: pl.BlockSpec(shp, lambda i: (0,) * len(shp))
    return pl.pallas_call(
        _node_body,
        grid=(_N // _RB,),
        in_specs=[
            pl.BlockSpec((_RB, _D), lambda i: (i, 0)),
            pl.BlockSpec((_RB, _D), lambda i: (i, 0)),
            pl.BlockSpec((_NC, _RB, _D), lambda i: (0, i, 0)),
            pl.BlockSpec((_NW, _RB, 1), lambda i: (0, i, 0)),
            full((_D, _D)), full((_D, _D)), full((1, _D)),
            full((_D, _D)), full((1, _D)),
            full((_D, _D)), full((1, _D)),
            full((_D, _D)), full((1, _D)),
            full((1, _D)), full((1, _D)),
        ],
        out_specs=pl.BlockSpec((_RB, _D), lambda i: (i, 0)),
        out_shape=jax.ShapeDtypeStruct((_N, _D), _f32),
    )(h, hn, msum2, cnt2, wna, wnb, bn1, wn2, bn2, wm1, bm1, wm2, bm2, g2, bt2)


def kernel(x, h, edge_index, We1, be1, We2, be2, Wn1, bn1, Wn2, bn2,
           Wm1, bm1, Wm2, bm2, g1, bt1, g2, bt2):
    ei = edge_index.astype(_i32)
    row = ei[0]
    col = ei[1]
    x0 = x[:, 0]
    x1 = x[:, 1]
    x2 = x[:, 2]

    w1a = We1[:_D]
    w1b = We1[_D:2 * _D]
    wd = We1[2 * _D].reshape(1, _D)
    r1 = lambda v: v.reshape(1, _D)

    hn, pa, pb = _ln_tables(h, r1(g1), r1(bt1), w1a, w1b, r1(be1))
    sa, d2 = _gather_sc(pa, pb, x0, x1, x2, row, col)
    m = _edge_mlp(sa, d2, wd, We2, r1(be2))

    z128 = jnp.zeros((_N, _D), _f32)
    zcnt = jnp.zeros((_NP + 16,), _f32)
    msum2, cntf = _scatter_sc(m, col, z128, zcnt)
    cnt2 = cntf.reshape(_NW, _NP)[:, :_N].reshape(_NW, _N, 1)

    return _node_mlp(h, hn, msum2, cnt2, Wn1[:_D], Wn1[_D:], r1(bn1),
                     Wn2, r1(bn2), Wm1, r1(bm1), Wm2, r1(bm2), r1(g2), r1(bt2))


# speedup vs baseline: 1.0129x; 1.0129x over previous
import functools

import jax
import jax.numpy as jnp
from jax import lax
from jax.experimental import pallas as pl
from jax.experimental.pallas import tpu as pltpu
from jax.experimental.pallas import tpu_sc as plsc

_N = 10000
_E = 320000
_D = 128
_NC = 2
_NS = 16
_NW = _NC * _NS
_EPW = _E // _NW
_CH = 80
_NCH = _EPW // _CH
_RB = 1000
_EB = 4000
_CW = 8
_NP = 10240

_f32 = jnp.float32
_bf16 = jnp.bfloat16


def _silu(z):
    return 0.5 * z * (1.0 + jnp.tanh(0.5 * z))
_i32 = jnp.int32


def _ln_tables_body(h_ref, g_ref, b_ref, wa_ref, wb_ref, bb_ref,
                    hn_ref, pa_ref, pb_ref):
    hb = h_ref[...]
    mu = jnp.mean(hb, axis=1, keepdims=True)
    ctr = hb - mu
    var = jnp.mean(ctr * ctr, axis=1, keepdims=True)
    hn = ctr * lax.rsqrt(var + 1e-5) * g_ref[...] + b_ref[...]
    hn_ref[...] = hn
    pa_ref[...] = jnp.dot(hn, wa_ref[...], preferred_element_type=_f32)
    pb_ref[...] = jnp.dot(hn, wb_ref[...], preferred_element_type=_f32) + bb_ref[...]


def _ln_tables(h, g1, bt1, w1a, w1b, be1):
    full = lambda shp: pl.BlockSpec(shp, lambda i: (0,) * len(shp))
    return pl.pallas_call(
        _ln_tables_body,
        grid=(_N // _RB,),
        in_specs=[
            pl.BlockSpec((_RB, _D), lambda i: (i, 0)),
            full((1, _D)), full((1, _D)),
            full((_D, _D)), full((_D, _D)), full((1, _D)),
        ],
        out_specs=[pl.BlockSpec((_RB, _D), lambda i: (i, 0))] * 3,
        out_shape=[jax.ShapeDtypeStruct((_N, _D), _f32)] * 3,
    )(h, g1, bt1, w1a, w1b, be1)


def _gather_sc(pa, pb, x0, x1, x2, row, col):
    mesh = plsc.VectorSubcoreMesh(core_axis_name="c", subcore_axis_name="s",
                                  num_cores=_NC, num_subcores=_NS)

    @functools.partial(
        pl.kernel,
        out_type=(jax.ShapeDtypeStruct((_E, _D), _f32),
                  jax.ShapeDtypeStruct((_E, _CW), _f32)),
        mesh=mesh,
        compiler_params=pltpu.CompilerParams(needs_layout_passes=False),
        scratch_types=[
            pltpu.VMEM((2, _CH), _i32),
            pltpu.VMEM((2, _CH), _i32),
            pltpu.VMEM((_CH, _D), _f32),
            pltpu.VMEM((_CH, _D), _f32),
            pltpu.VMEM((_CH, _D), _f32),
            pltpu.VMEM((_CH, _D), _f32),
            pltpu.VMEM((2, _CH, _CW), _f32),
            pltpu.VMEM((_N,), _f32),
            pltpu.VMEM((_N,), _f32),
            pltpu.VMEM((_N,), _f32),
            pltpu.SemaphoreType.DMA,
            pltpu.SemaphoreType.DMA,
            pltpu.SemaphoreType.DMA,
        ],
    )
    def k(pa_h, pb_h, x0_h, x1_h, x2_h, row_h, col_h,
          sa_h, d2_h,
          idxr2, idxc2, bufa0, bufa1, bufb0, bufb1, d2b2, x0v, x1v, x2v,
          semg0, semg1, semw):
        semg = (semg0, semg1)
        c = lax.axis_index("c")
        s = lax.axis_index("s")
        ebase = (c * _NS + s) * _EPW
        pltpu.sync_copy(x0_h, x0v)
        pltpu.sync_copy(x1_h, x1v)
        pltpu.sync_copy(x2_h, x2v)
        lanes = lax.iota(_i32, 16)
        zeros16 = jnp.zeros((16,), _i32)
        bufa = (bufa0, bufa1)
        bufb = (bufb0, bufb1)

        def load_idx(cc, S):
            base = ebase + cc * _CH
            pltpu.sync_copy(row_h.at[pl.ds(base, _CH)], idxr2.at[S])
            pltpu.sync_copy(col_h.at[pl.ds(base, _CH)], idxc2.at[S])

        def start_gathers(S):
            pltpu.async_copy(pa_h.at[idxr2.at[S]], bufa[S], semg[S])
            pltpu.async_copy(pb_h.at[idxc2.at[S]], bufb[S], semg[S])

        def drain_gathers(S):
            pltpu.make_async_copy(pa_h.at[idxr2.at[S]], bufa[S], semg[S]).wait()
            pltpu.make_async_copy(pb_h.at[idxc2.at[S]], bufb[S], semg[S]).wait()

        def start_wb(cc, S):
            base = ebase + cc * _CH
            pltpu.async_copy(bufa[S], sa_h.at[pl.ds(base, _CH)], semw)
            pltpu.async_copy(d2b2.at[S], d2_h.at[pl.ds(base, _CH)], semw)

        def drain_wb(cc, S):
            base = ebase + cc * _CH
            pltpu.make_async_copy(bufa[S], sa_h.at[pl.ds(base, _CH)], semw).wait()
            pltpu.make_async_copy(d2b2.at[S], d2_h.at[pl.ds(base, _CH)], semw).wait()

        def add_rows(S):
            def row_add(r, carry2):
                for l in range(_D // 16):
                    sl = pl.ds(l * 16, 16)
                    bufa[S][r, sl] = bufa[S][r, sl] + bufb[S][r, sl]
                return carry2

            lax.fori_loop(0, _CH, row_add, 0)

        def compute_d2(S):
            def dist_group(g, carry2):
                ir = idxr2[S, pl.ds(g * 16, 16)]
                ic = idxc2[S, pl.ds(g * 16, 16)]
                dx = plsc.load_gather(x0v, [ir]) - plsc.load_gather(x0v, [ic])
                dy = plsc.load_gather(x1v, [ir]) - plsc.load_gather(x1v, [ic])
                dz = plsc.load_gather(x2v, [ir]) - plsc.load_gather(x2v, [ic])
                d2v = dx * dx + dy * dy + dz * dz
                plsc.store_scatter(d2b2.at[S], [g * 16 + lanes, zeros16], d2v)
                return carry2

            lax.fori_loop(0, _CH // 16, dist_group, 0)

        def phase(cc, S, Sp):
            @pl.when(cc < _NCH)
            def _():
                @pl.when(cc > 0)
                def _():
                    drain_wb(cc - 1, Sp)

                @pl.when(cc + 1 < _NCH)
                def _():
                    load_idx(cc + 1, Sp)
                    start_gathers(Sp)

                drain_gathers(S)
                compute_d2(S)
                add_rows(S)
                start_wb(cc, S)

        load_idx(0, 0)
        start_gathers(0)

        def body(j, carry):
            phase(2 * j, 0, 1)
            phase(2 * j + 1, 1, 0)
            return carry

        lax.fori_loop(0, (_NCH + 2) // 2, body, 0)
        drain_wb(_NCH - 1, (_NCH - 1) % 2)

    return k(pa, pb, x0, x1, x2, row, col)


def _edge_mlp_body(sa_ref, d2_ref, wd_ref, w2_ref, b2_ref, m_ref):
    dist = jnp.sqrt(d2_ref[...][:, 0:1])
    z = (sa_ref[...] + dist * wd_ref[...]).astype(_bf16)
    m1 = _silu(z)
    z2 = ((jnp.dot(m1, w2_ref[...], preferred_element_type=_f32)
           + b2_ref[...]).astype(_bf16))
    m_ref[...] = _silu(z2).astype(_f32)


def _edge_mlp(sa, d2, wd, w2, b2):
    full = lambda shp: pl.BlockSpec(shp, lambda i: (0,) * len(shp))
    return pl.pallas_call(
        _edge_mlp_body,
        grid=(_E // _EB,),
        in_specs=[
            pl.BlockSpec((_EB, _D), lambda i: (i, 0)),
            pl.BlockSpec((_EB, _CW), lambda i: (i, 0)),
            full((1, _D)), full((_D, _D)), full((1, _D)),
        ],
        out_specs=pl.BlockSpec((_EB, _D), lambda i: (i, 0)),
        out_shape=jax.ShapeDtypeStruct((_E, _D), _f32),
    )(sa, d2, wd, w2.astype(_bf16), b2)


def _scatter_sc(m, col, z128, zcnt):
    mesh = plsc.VectorSubcoreMesh(core_axis_name="c", subcore_axis_name="s",
                                  num_cores=_NC, num_subcores=_NS)

    @functools.partial(
        pl.kernel,
        out_type=(jax.ShapeDtypeStruct((_NC, _N, _D), _f32),
                  jax.ShapeDtypeStruct((_NW * _NP,), _f32)),
        mesh=mesh,
        compiler_params=pltpu.CompilerParams(needs_layout_passes=False),
        scratch_types=[
            pltpu.VMEM((2, _CH), _i32),
            pltpu.VMEM((_CH, _D), _f32),
            pltpu.VMEM((_CH, _D), _f32),
            pltpu.VMEM((_NP + 16,), _f32),
            pltpu.VMEM_SHARED((_N, _D), _f32),
            pltpu.SemaphoreType.DMA,
            pltpu.SemaphoreType.DMA,
            pltpu.SemaphoreType.DMA,
        ],
    )
    def k(m_h, col_h, z128_h, zcnt_h, msum_h, cnt_h, idx2, data0, data1,
          cntv, msh, seml, semsc0, semsc1):
        semsc = (semsc0, semsc1)
        data = (data0, data1)
        c = lax.axis_index("c")
        s = lax.axis_index("s")
        wid = c * _NS + s
        ebase = wid * _EPW
        pltpu.sync_copy(zcnt_h, cntv)
        lanes = lax.iota(_i32, 16)

        def count_chunk(S):
            def count16(g, carry2):
                ivvec = idx2[S, pl.ds(g * 16, 16)]
                for jj in range(16):
                    iv = ivvec[jj]
                    cbase = lax.shift_left(lax.shift_right_logical(iv, 3), 3)
                    lane = iv - cbase
                    cntv[pl.ds(cbase, 16)] = (cntv[pl.ds(cbase, 16)]
                                              + (lanes == lane).astype(_f32))
                return carry2

            lax.fori_loop(0, _CH // 16, count16, 0)

        def start_loads(cc, S):
            base = ebase + cc * _CH
            pltpu.async_copy(col_h.at[pl.ds(base, _CH)], idx2.at[S], seml)
            pltpu.async_copy(m_h.at[pl.ds(base, _CH)], data[S], seml)

        def drain_loads(cc, S):
            base = ebase + cc * _CH
            pltpu.make_async_copy(col_h.at[pl.ds(base, _CH)], idx2.at[S], seml).wait()
            pltpu.make_async_copy(m_h.at[pl.ds(base, _CH)], data[S], seml).wait()

        def start_scatter(S):
            pltpu.async_copy(data[S], msh.at[idx2.at[S]], semsc[S], add=True)

        def drain_scatter(S):
            pltpu.make_async_copy(data[S], msh.at[idx2.at[S]], semsc[S]).wait()

        @pl.when(s == 0)
        def _():
            pltpu.sync_copy(z128_h, msh)

        plsc.subcore_barrier()
        start_loads(0, 0)

        def phase(cc, S, Sp):
            @pl.when(cc < _NCH)
            def _():
                drain_loads(cc, S)

                @pl.when(cc > 0)
                def _():
                    drain_scatter(Sp)

                @pl.when(cc + 1 < _NCH)
                def _():
                    start_loads(cc + 1, Sp)

                start_scatter(S)
                count_chunk(S)

        def body(j, carry):
            phase(2 * j, 0, 1)
            phase(2 * j + 1, 1, 0)
            return carry

        lax.fori_loop(0, (_NCH + 2) // 2, body, 0)
        drain_scatter((_NCH - 1) % 2)
        pltpu.sync_copy(cntv.at[pl.ds(0, _NP)], cnt_h.at[pl.ds(wid * _NP, _NP)])
        plsc.subcore_barrier()
        rpt0 = 624

        @pl.when(s < _NS - 1)
        def _():
            rb = s * rpt0
            pltpu.sync_copy(msh.at[pl.ds(rb, rpt0)], msum_h.at[c, pl.ds(rb, rpt0)])

        @pl.when(s == _NS - 1)
        def _():
            rb = (_NS - 1) * rpt0
            rpt1 = _N - rb
            pltpu.sync_copy(msh.at[pl.ds(rb, rpt1)], msum_h.at[c, pl.ds(rb, rpt1)])

    return k(m, col, z128, zcnt)


def _node_body(h_ref, hn_ref, ms_ref, ct_ref, wna_ref, wnb_ref, bn1_ref,
               wn2_ref, bn2_ref, wm1_ref, bm1_ref, wm2_ref, bm2_ref,
               g2_ref, b2t_ref, out_ref):
    ms = ms_ref[0] + ms_ref[1]
    cnt = jnp.sum(ct_ref[...], axis=0)[:, 0:1]
    maggr = ms / jnp.maximum(cnt, 1.0)
    hn = hn_ref[...]
    bf = lambda v: v.astype(_bf16)
    z = (jnp.dot(bf(hn), bf(wna_ref[...]), preferred_element_type=_f32)
         + jnp.dot(bf(maggr), bf(wnb_ref[...]), preferred_element_type=_f32)
         + bn1_ref[...])
    a = _silu(z)
    h_delta = (jnp.dot(bf(a), bf(wn2_ref[...]), preferred_element_type=_f32)
               + bn2_ref[...])
    h1 = h_ref[...] + hn + h_delta
    mu = jnp.mean(h1, axis=1, keepdims=True)
    ctr = h1 - mu
    var = jnp.mean(ctr * ctr, axis=1, keepdims=True)
    hn2 = ctr * lax.rsqrt(var + 1e-5) * g2_ref[...] + b2t_ref[...]
    z2 = (jnp.dot(bf(hn2), bf(wm1_ref[...]), preferred_element_type=_f32)
          + bm1_ref[...])
    a2 = _silu(z2)
    out_ref[...] = (h1 + jnp.dot(bf(a2), bf(wm2_ref[...]), preferred_element_type=_f32)
                    + bm2_ref[...])


def _node_mlp(h, hn, msum2, cnt2, wna, wnb, bn1, wn2, bn2, wm1, bm1, wm2, bm2, g2, bt2):
    full = lambda shp: pl.BlockSpec(shp, lambda i: (0,) * len(shp))
    return pl.pallas_call(
        _node_body,
        grid=(_N // _RB,),
        in_specs=[
            pl.BlockSpec((_RB, _D), lambda i: (i, 0)),
            pl.BlockSpec((_RB, _D), lambda i: (i, 0)),
            pl.BlockSpec((_NC, _RB, _D), lambda i: (0, i, 0)),
            pl.BlockSpec((_NW, _RB, 1), lambda i: (0, i, 0)),
            full((_D, _D)), full((_D, _D)), full((1, _D)),
            full((_D, _D)), full((1, _D)),
            full((_D, _D)), full((1, _D)),
            full((_D, _D)), full((1, _D)),
            full((1, _D)), full((1, _D)),
        ],
        out_specs=pl.BlockSpec((_RB, _D), lambda i: (i, 0)),
        out_shape=jax.ShapeDtypeStruct((_N, _D), _f32),
    )(h, hn, msum2, cnt2, wna, wnb, bn1, wn2, bn2, wm1, bm1, wm2, bm2, g2, bt2)


def kernel(x, h, edge_index, We1, be1, We2, be2, Wn1, bn1, Wn2, bn2,
           Wm1, bm1, Wm2, bm2, g1, bt1, g2, bt2):
    ei = edge_index.astype(_i32)
    row = ei[0]
    col = ei[1]
    x0 = x[:, 0]
    x1 = x[:, 1]
    x2 = x[:, 2]

    w1a = We1[:_D]
    w1b = We1[_D:2 * _D]
    wd = We1[2 * _D].reshape(1, _D)
    r1 = lambda v: v.reshape(1, _D)

    hn, pa, pb = _ln_tables(h, r1(g1), r1(bt1), w1a, w1b, r1(be1))
    sa, d2 = _gather_sc(pa, pb, x0, x1, x2, row, col)
    m = _edge_mlp(sa, d2, wd, We2, r1(be2))

    z128 = jnp.zeros((_N, _D), _f32)
    zcnt = jnp.zeros((_NP + 16,), _f32)
    msum2, cntf = _scatter_sc(m, col, z128, zcnt)
    cnt2 = cntf.reshape(_NW, _NP)[:, :_N].reshape(_NW, _N, 1)

    return _node_mlp(h, hn, msum2, cnt2, Wn1[:_D], Wn1[_D:], r1(bn1),
                     Wn2, r1(bn2), Wm1, r1(bm1), Wm2, r1(bm2), r1(g2), r1(bt2))


# speedup vs baseline: 1.1480x; 1.1333x over previous
import functools

import jax
import jax.numpy as jnp
from jax import lax
from jax.experimental import pallas as pl
from jax.experimental.pallas import tpu as pltpu
from jax.experimental.pallas import tpu_sc as plsc

_N = 10000
_E = 320000
_D = 128
_NC = 2
_NS = 16
_NW = _NC * _NS
_EPW = _E // _NW
_CH = 80
_NCH = _EPW // _CH
_RB = 1000
_EB = 4000
_CW = 8
_NP = 10240

_f32 = jnp.float32
_bf16 = jnp.bfloat16


def _silu(z):
    return 0.5 * z * (1.0 + jnp.tanh(0.5 * z))
_i32 = jnp.int32


def _ln_tables_body(h_ref, g_ref, b_ref, wa_ref, wb_ref, bb_ref,
                    hn_ref, pa_ref, pb_ref):
    hb = h_ref[...]
    mu = jnp.mean(hb, axis=1, keepdims=True)
    ctr = hb - mu
    var = jnp.mean(ctr * ctr, axis=1, keepdims=True)
    hn = ctr * lax.rsqrt(var + 1e-5) * g_ref[...] + b_ref[...]
    hn_ref[...] = hn
    pa_ref[...] = jnp.dot(hn, wa_ref[...], preferred_element_type=_f32)
    pb_ref[...] = jnp.dot(hn, wb_ref[...], preferred_element_type=_f32) + bb_ref[...]


def _ln_tables(h, g1, bt1, w1a, w1b, be1):
    full = lambda shp: pl.BlockSpec(shp, lambda i: (0,) * len(shp))
    return pl.pallas_call(
        _ln_tables_body,
        grid=(_N // _RB,),
        in_specs=[
            pl.BlockSpec((_RB, _D), lambda i: (i, 0)),
            full((1, _D)), full((1, _D)),
            full((_D, _D)), full((_D, _D)), full((1, _D)),
        ],
        out_specs=[pl.BlockSpec((_RB, _D), lambda i: (i, 0))] * 3,
        out_shape=[jax.ShapeDtypeStruct((_N, _D), _f32)] * 3,
    )(h, g1, bt1, w1a, w1b, be1)


def _gather_sc(pa, pb, x0, x1, x2, row, col):
    mesh = plsc.VectorSubcoreMesh(core_axis_name="c", subcore_axis_name="s",
                                  num_cores=_NC, num_subcores=_NS)

    @functools.partial(
        pl.kernel,
        out_type=(jax.ShapeDtypeStruct((_E, _D), _f32),
                  jax.ShapeDtypeStruct((_E, _CW), _f32)),
        mesh=mesh,
        compiler_params=pltpu.CompilerParams(needs_layout_passes=False),
        scratch_types=[
            pltpu.VMEM((2, _CH), _i32),
            pltpu.VMEM((2, _CH), _i32),
            pltpu.VMEM((_CH, _D), _f32),
            pltpu.VMEM((_CH, _D), _f32),
            pltpu.VMEM((_CH, _D), _f32),
            pltpu.VMEM((_CH, _D), _f32),
            pltpu.VMEM((2, _CH, _CW), _f32),
            pltpu.VMEM((_N,), _f32),
            pltpu.VMEM((_N,), _f32),
            pltpu.VMEM((_N,), _f32),
            pltpu.SemaphoreType.DMA,
            pltpu.SemaphoreType.DMA,
            pltpu.SemaphoreType.DMA,
        ],
    )
    def k(pa_h, pb_h, x0_h, x1_h, x2_h, row_h, col_h,
          sa_h, d2_h,
          idxr2, idxc2, bufa0, bufa1, bufb0, bufb1, d2b2, x0v, x1v, x2v,
          semg0, semg1, semw):
        semg = (semg0, semg1)
        c = lax.axis_index("c")
        s = lax.axis_index("s")
        ebase = (c * _NS + s) * _EPW
        pltpu.sync_copy(x0_h, x0v)
        pltpu.sync_copy(x1_h, x1v)
        pltpu.sync_copy(x2_h, x2v)
        lanes = lax.iota(_i32, 16)
        zeros16 = jnp.zeros((16,), _i32)
        bufa = (bufa0, bufa1)
        bufb = (bufb0, bufb1)

        def load_idx(cc, S):
            base = ebase + cc * _CH
            pltpu.sync_copy(row_h.at[pl.ds(base, _CH)], idxr2.at[S])
            pltpu.sync_copy(col_h.at[pl.ds(base, _CH)], idxc2.at[S])

        def start_gathers(S):
            pltpu.async_copy(pa_h.at[idxr2.at[S]], bufa[S], semg[S])
            pltpu.async_copy(pb_h.at[idxc2.at[S]], bufb[S], semg[S])

        def drain_gathers(S):
            pltpu.make_async_copy(pa_h.at[idxr2.at[S]], bufa[S], semg[S]).wait()
            pltpu.make_async_copy(pb_h.at[idxc2.at[S]], bufb[S], semg[S]).wait()

        def start_wb(cc, S):
            base = ebase + cc * _CH
            pltpu.async_copy(bufa[S], sa_h.at[pl.ds(base, _CH)], semw)
            pltpu.async_copy(d2b2.at[S], d2_h.at[pl.ds(base, _CH)], semw)

        def drain_wb(cc, S):
            base = ebase + cc * _CH
            pltpu.make_async_copy(bufa[S], sa_h.at[pl.ds(base, _CH)], semw).wait()
            pltpu.make_async_copy(d2b2.at[S], d2_h.at[pl.ds(base, _CH)], semw).wait()

        def add_rows(S):
            def row_add(r, carry2):
                for l in range(_D // 16):
                    sl = pl.ds(l * 16, 16)
                    bufa[S][r, sl] = bufa[S][r, sl] + bufb[S][r, sl]
                return carry2

            lax.fori_loop(0, _CH, row_add, 0)

        def compute_d2(S):
            def dist_group(g, carry2):
                ir = idxr2[S, pl.ds(g * 16, 16)]
                ic = idxc2[S, pl.ds(g * 16, 16)]
                dx = plsc.load_gather(x0v, [ir]) - plsc.load_gather(x0v, [ic])
                dy = plsc.load_gather(x1v, [ir]) - plsc.load_gather(x1v, [ic])
                dz = plsc.load_gather(x2v, [ir]) - plsc.load_gather(x2v, [ic])
                d2v = dx * dx + dy * dy + dz * dz
                plsc.store_scatter(d2b2.at[S], [g * 16 + lanes, zeros16], d2v)
                return carry2

            lax.fori_loop(0, _CH // 16, dist_group, 0)

        def phase(cc, S, Sp):
            @pl.when(cc < _NCH)
            def _():
                @pl.when(cc > 0)
                def _():
                    drain_wb(cc - 1, Sp)

                @pl.when(cc + 1 < _NCH)
                def _():
                    load_idx(cc + 1, Sp)
                    start_gathers(Sp)

                drain_gathers(S)
                compute_d2(S)
                add_rows(S)
                start_wb(cc, S)

        load_idx(0, 0)
        start_gathers(0)

        def body(j, carry):
            phase(2 * j, 0, 1)
            phase(2 * j + 1, 1, 0)
            return carry

        lax.fori_loop(0, (_NCH + 2) // 2, body, 0)
        drain_wb(_NCH - 1, (_NCH - 1) % 2)

    return k(pa, pb, x0, x1, x2, row, col)


def _edge_mlp_body(sa_ref, d2_ref, wd_ref, w2_ref, b2_ref, m_ref):
    dist = jnp.sqrt(d2_ref[...][:, 0:1])
    z = (sa_ref[...] + dist * wd_ref[...]).astype(_bf16)
    m1 = _silu(z)
    z2 = ((jnp.dot(m1, w2_ref[...], preferred_element_type=_f32)
           + b2_ref[...]).astype(_bf16))
    m_ref[...] = _silu(z2).astype(_f32)


def _edge_mlp(sa, d2, wd, w2, b2):
    full = lambda shp: pl.BlockSpec(shp, lambda i: (0,) * len(shp))
    return pl.pallas_call(
        _edge_mlp_body,
        grid=(_E // _EB,),
        in_specs=[
            pl.BlockSpec((_EB, _D), lambda i: (i, 0)),
            pl.BlockSpec((_EB, _CW), lambda i: (i, 0)),
            full((1, _D)), full((_D, _D)), full((1, _D)),
        ],
        out_specs=pl.BlockSpec((_EB, _D), lambda i: (i, 0)),
        out_shape=jax.ShapeDtypeStruct((_E, _D), _f32),
    )(sa, d2, wd, w2.astype(_bf16), b2)


def _scatter_sc(m, col, z128, zcnt):
    mesh = plsc.VectorSubcoreMesh(core_axis_name="c", subcore_axis_name="s",
                                  num_cores=_NC, num_subcores=_NS)

    @functools.partial(
        pl.kernel,
        out_type=(jax.ShapeDtypeStruct((_NC, _N, _D), _f32),
                  jax.ShapeDtypeStruct((_NW * _NP,), _f32)),
        mesh=mesh,
        compiler_params=pltpu.CompilerParams(needs_layout_passes=False),
        scratch_types=[
            pltpu.VMEM((2, _CH), _i32),
            pltpu.VMEM((_CH, _D), _f32),
            pltpu.VMEM((_CH, _D), _f32),
            pltpu.VMEM((_NP + 16,), _f32),
            pltpu.VMEM_SHARED((_N, _D), _f32),
            pltpu.SemaphoreType.DMA,
            pltpu.SemaphoreType.DMA,
            pltpu.SemaphoreType.DMA,
        ],
    )
    def k(m_h, col_h, z128_h, zcnt_h, msum_h, cnt_h, idx2, data0, data1,
          cntv, msh, seml, semsc0, semsc1):
        semsc = (semsc0, semsc1)
        data = (data0, data1)
        c = lax.axis_index("c")
        s = lax.axis_index("s")
        wid = c * _NS + s
        ebase = wid * _EPW
        pltpu.sync_copy(zcnt_h, cntv)
        lanes = lax.iota(_i32, 16)

        def count_chunk(S):
            def count16(g, carry2):
                ivvec = idx2[S, pl.ds(g * 16, 16)]
                for jj in range(16):
                    iv = ivvec[jj]
                    cbase = lax.shift_left(lax.shift_right_logical(iv, 3), 3)
                    lane = iv - cbase
                    cntv[pl.ds(cbase, 16)] = (cntv[pl.ds(cbase, 16)]
                                              + (lanes == lane).astype(_f32))
                return carry2

            lax.fori_loop(0, _CH // 16, count16, 0)

        def start_loads(cc, S):
            base = ebase + cc * _CH
            pltpu.async_copy(col_h.at[pl.ds(base, _CH)], idx2.at[S], seml)
            pltpu.async_copy(m_h.at[pl.ds(base, _CH)], data[S], seml)

        def drain_loads(cc, S):
            base = ebase + cc * _CH
            pltpu.make_async_copy(col_h.at[pl.ds(base, _CH)], idx2.at[S], seml).wait()
            pltpu.make_async_copy(m_h.at[pl.ds(base, _CH)], data[S], seml).wait()

        def start_scatter(S):
            pltpu.async_copy(data[S], msh.at[idx2.at[S]], semsc[S], add=True)

        def drain_scatter(S):
            pltpu.make_async_copy(data[S], msh.at[idx2.at[S]], semsc[S]).wait()

        @pl.when(s == 0)
        def _():
            pltpu.sync_copy(z128_h, msh)

        plsc.subcore_barrier()
        start_loads(0, 0)

        def phase(cc, S, Sp):
            @pl.when(cc < _NCH)
            def _():
                drain_loads(cc, S)

                @pl.when(cc > 0)
                def _():
                    drain_scatter(Sp)

                @pl.when(cc + 1 < _NCH)
                def _():
                    start_loads(cc + 1, Sp)

                start_scatter(S)
                count_chunk(S)

        def body(j, carry):
            phase(2 * j, 0, 1)
            phase(2 * j + 1, 1, 0)
            return carry

        lax.fori_loop(0, (_NCH + 2) // 2, body, 0)
        drain_scatter((_NCH - 1) % 2)
        pltpu.sync_copy(cntv.at[pl.ds(0, _NP)], cnt_h.at[pl.ds(wid * _NP, _NP)])
        plsc.subcore_barrier()
        rpt0 = 624

        @pl.when(s < _NS - 1)
        def _():
            rb = s * rpt0
            pltpu.sync_copy(msh.at[pl.ds(rb, rpt0)], msum_h.at[c, pl.ds(rb, rpt0)])

        @pl.when(s == _NS - 1)
        def _():
            rb = (_NS - 1) * rpt0
            rpt1 = _N - rb
            pltpu.sync_copy(msh.at[pl.ds(rb, rpt1)], msum_h.at[c, pl.ds(rb, rpt1)])

    return k(m, col, z128, zcnt)


def _cnt_transpose_body(ct_ref, eye_ref, out_ref):
    s = jnp.sum(ct_ref[...], axis=0)
    pieces = [lax.dot_general(eye_ref[...], s[k:k + 1, :],
                              (((1,), (1,)), ((), ())),
                              preferred_element_type=_f32)
              for k in range(8)]
    out_ref[...] = jnp.concatenate(pieces, axis=0)


def _cnt_transpose(cnt3, eye):
    return pl.pallas_call(
        _cnt_transpose_body,
        grid=(_NP // 1024,),
        in_specs=[
            pl.BlockSpec((_NW, 8, 128), lambda i: (0, i, 0)),
            pl.BlockSpec((128, 128), lambda i: (0, 0)),
        ],
        out_specs=pl.BlockSpec((1024, 1), lambda i: (i, 0)),
        out_shape=jax.ShapeDtypeStruct((_NP, 1), _f32),
    )(cnt3, eye)


def _node_body(h_ref, hn_ref, ms_ref, ct_ref, wna_ref, wnb_ref, bn1_ref,
               wn2_ref, bn2_ref, wm1_ref, bm1_ref, wm2_ref, bm2_ref,
               g2_ref, b2t_ref, out_ref):
    ms = ms_ref[0] + ms_ref[1]
    maggr = ms / jnp.maximum(ct_ref[...], 1.0)
    hn = hn_ref[...]
    bf = lambda v: v.astype(_bf16)
    z = (jnp.dot(bf(hn), bf(wna_ref[...]), preferred_element_type=_f32)
         + jnp.dot(bf(maggr), bf(wnb_ref[...]), preferred_element_type=_f32)
         + bn1_ref[...])
    a = _silu(z)
    h_delta = (jnp.dot(bf(a), bf(wn2_ref[...]), preferred_element_type=_f32)
               + bn2_ref[...])
    h1 = h_ref[...] + hn + h_delta
    mu = jnp.mean(h1, axis=1, keepdims=True)
    ctr = h1 - mu
    var = jnp.mean(ctr * ctr, axis=1, keepdims=True)
    hn2 = ctr * lax.rsqrt(var + 1e-5) * g2_ref[...] + b2t_ref[...]
    z2 = (jnp.dot(bf(hn2), bf(wm1_ref[...]), preferred_element_type=_f32)
          + bm1_ref[...])
    a2 = _silu(z2)
    out_ref[...] = (h1 + jnp.dot(bf(a2), bf(wm2_ref[...]), preferred_element_type=_f32)
                    + bm2_ref[...])


def _node_mlp(h, hn, msum2, cnt2, wna, wnb, bn1, wn2, bn2, wm1, bm1, wm2, bm2, g2, bt2):
    full = lambda shp: pl.BlockSpec(shp, lambda i: (0,) * len(shp))
    return pl.pallas_call(
        _node_body,
        grid=(_N // _RB,),
        in_specs=[
            pl.BlockSpec((_RB, _D), lambda i: (i, 0)),
            pl.BlockSpec((_RB, _D), lambda i: (i, 0)),
            pl.BlockSpec((_NC, _RB, _D), lambda i: (0, i, 0)),
            pl.BlockSpec((_RB, 1), lambda i: (i, 0)),
            full((_D, _D)), full((_D, _D)), full((1, _D)),
            full((_D, _D)), full((1, _D)),
            full((_D, _D)), full((1, _D)),
            full((_D, _D)), full((1, _D)),
            full((1, _D)), full((1, _D)),
        ],
        out_specs=pl.BlockSpec((_RB, _D), lambda i: (i, 0)),
        out_shape=jax.ShapeDtypeStruct((_N, _D), _f32),
    )(h, hn, msum2, cnt2, wna, wnb, bn1, wn2, bn2, wm1, bm1, wm2, bm2, g2, bt2)


def kernel(x, h, edge_index, We1, be1, We2, be2, Wn1, bn1, Wn2, bn2,
           Wm1, bm1, Wm2, bm2, g1, bt1, g2, bt2):
    ei = edge_index.astype(_i32)
    row = ei[0]
    col = ei[1]
    x0 = x[:, 0]
    x1 = x[:, 1]
    x2 = x[:, 2]

    w1a = We1[:_D]
    w1b = We1[_D:2 * _D]
    wd = We1[2 * _D].reshape(1, _D)
    r1 = lambda v: v.reshape(1, _D)

    hn, pa, pb = _ln_tables(h, r1(g1), r1(bt1), w1a, w1b, r1(be1))
    sa, d2 = _gather_sc(pa, pb, x0, x1, x2, row, col)
    m = _edge_mlp(sa, d2, wd, We2, r1(be2))

    z128 = jnp.zeros((_N, _D), _f32)
    zcnt = jnp.zeros((_NP + 16,), _f32)
    msum2, cntf = _scatter_sc(m, col, z128, zcnt)
    eye128 = jnp.eye(128, dtype=_f32)
    cntcol = _cnt_transpose(cntf.reshape(_NW, _NP // 128, 128), eye128)[:_N]

    return _node_mlp(h, hn, msum2, cntcol, Wn1[:_D], Wn1[_D:], r1(bn1),
                     Wn2, r1(bn2), Wm1, r1(bm1), Wm2, r1(bm2), r1(g2), r1(bt2))


# speedup vs baseline: 1.1502x; 1.0019x over previous
import functools

import jax
import jax.numpy as jnp
from jax import lax
from jax.experimental import pallas as pl
from jax.experimental.pallas import tpu as pltpu
from jax.experimental.pallas import tpu_sc as plsc

_N = 10000
_E = 320000
_D = 128
_NC = 2
_NS = 16
_NW = _NC * _NS
_EPW = _E // _NW
_CH = 80
_NCH = _EPW // _CH
_RB = 1000
_EB = 4000
_CW = 8
_NP = 10240

_f32 = jnp.float32
_bf16 = jnp.bfloat16


def _silu(z):
    return 0.5 * z * (1.0 + jnp.tanh(0.5 * z))
_i32 = jnp.int32


def _ln_tables_body(h_ref, g_ref, b_ref, wa_ref, wb_ref, bb_ref,
                    hn_ref, pa_ref, pb_ref):
    hb = h_ref[...]
    mu = jnp.mean(hb, axis=1, keepdims=True)
    ctr = hb - mu
    var = jnp.mean(ctr * ctr, axis=1, keepdims=True)
    hn = ctr * lax.rsqrt(var + 1e-5) * g_ref[...] + b_ref[...]
    hn_ref[...] = hn
    pa_ref[...] = jnp.dot(hn, wa_ref[...], preferred_element_type=_f32)
    pb_ref[...] = jnp.dot(hn, wb_ref[...], preferred_element_type=_f32) + bb_ref[...]


def _ln_tables(h, g1, bt1, w1a, w1b, be1):
    full = lambda shp: pl.BlockSpec(shp, lambda i: (0,) * len(shp))
    return pl.pallas_call(
        _ln_tables_body,
        grid=(_N // _RB,),
        in_specs=[
            pl.BlockSpec((_RB, _D), lambda i: (i, 0)),
            full((1, _D)), full((1, _D)),
            full((_D, _D)), full((_D, _D)), full((1, _D)),
        ],
        out_specs=[pl.BlockSpec((_RB, _D), lambda i: (i, 0))] * 3,
        out_shape=[jax.ShapeDtypeStruct((_N, _D), _f32)] * 3,
    )(h, g1, bt1, w1a, w1b, be1)


def _gather_sc(pa, pb, x0, x1, x2, row, col):
    mesh = plsc.VectorSubcoreMesh(core_axis_name="c", subcore_axis_name="s",
                                  num_cores=_NC, num_subcores=_NS)

    @functools.partial(
        pl.kernel,
        out_type=(jax.ShapeDtypeStruct((_E, _D), _f32),
                  jax.ShapeDtypeStruct((_E, _CW), _f32)),
        mesh=mesh,
        compiler_params=pltpu.CompilerParams(needs_layout_passes=False),
        scratch_types=[
            pltpu.VMEM((2, _CH), _i32),
            pltpu.VMEM((2, _CH), _i32),
            pltpu.VMEM((_CH, _D), _f32),
            pltpu.VMEM((_CH, _D), _f32),
            pltpu.VMEM((_CH, _D), _f32),
            pltpu.VMEM((_CH, _D), _f32),
            pltpu.VMEM((2, _CH, _CW), _f32),
            pltpu.VMEM((_N,), _f32),
            pltpu.VMEM((_N,), _f32),
            pltpu.VMEM((_N,), _f32),
            pltpu.SemaphoreType.DMA,
            pltpu.SemaphoreType.DMA,
            pltpu.SemaphoreType.DMA,
        ],
    )
    def k(pa_h, pb_h, x0_h, x1_h, x2_h, row_h, col_h,
          sa_h, d2_h,
          idxr2, idxc2, bufa0, bufa1, bufb0, bufb1, d2b2, x0v, x1v, x2v,
          semg0, semg1, semw):
        semg = (semg0, semg1)
        c = lax.axis_index("c")
        s = lax.axis_index("s")
        ebase = (c * _NS + s) * _EPW
        pltpu.sync_copy(x0_h, x0v)
        pltpu.sync_copy(x1_h, x1v)
        pltpu.sync_copy(x2_h, x2v)
        lanes = lax.iota(_i32, 16)
        zeros16 = jnp.zeros((16,), _i32)
        bufa = (bufa0, bufa1)
        bufb = (bufb0, bufb1)

        def load_idx(cc, S):
            base = ebase + cc * _CH
            pltpu.sync_copy(row_h.at[pl.ds(base, _CH)], idxr2.at[S])
            pltpu.sync_copy(col_h.at[pl.ds(base, _CH)], idxc2.at[S])

        def start_gathers(S):
            pltpu.async_copy(pa_h.at[idxr2.at[S]], bufa[S], semg[S])
            pltpu.async_copy(pb_h.at[idxc2.at[S]], bufb[S], semg[S])

        def drain_gathers(S):
            pltpu.make_async_copy(pa_h.at[idxr2.at[S]], bufa[S], semg[S]).wait()
            pltpu.make_async_copy(pb_h.at[idxc2.at[S]], bufb[S], semg[S]).wait()

        def start_wb(cc, S):
            base = ebase + cc * _CH
            pltpu.async_copy(bufa[S], sa_h.at[pl.ds(base, _CH)], semw)
            pltpu.async_copy(d2b2.at[S], d2_h.at[pl.ds(base, _CH)], semw)

        def drain_wb(cc, S):
            base = ebase + cc * _CH
            pltpu.make_async_copy(bufa[S], sa_h.at[pl.ds(base, _CH)], semw).wait()
            pltpu.make_async_copy(d2b2.at[S], d2_h.at[pl.ds(base, _CH)], semw).wait()

        def add_rows(S):
            def row_add(r, carry2):
                for l in range(_D // 16):
                    sl = pl.ds(l * 16, 16)
                    bufa[S][r, sl] = bufa[S][r, sl] + bufb[S][r, sl]
                return carry2

            lax.fori_loop(0, _CH, row_add, 0)

        def compute_d2(S):
            def dist_group(g, carry2):
                ir = idxr2[S, pl.ds(g * 16, 16)]
                ic = idxc2[S, pl.ds(g * 16, 16)]
                dx = plsc.load_gather(x0v, [ir]) - plsc.load_gather(x0v, [ic])
                dy = plsc.load_gather(x1v, [ir]) - plsc.load_gather(x1v, [ic])
                dz = plsc.load_gather(x2v, [ir]) - plsc.load_gather(x2v, [ic])
                d2v = dx * dx + dy * dy + dz * dz
                plsc.store_scatter(d2b2.at[S], [g * 16 + lanes, zeros16], d2v)
                return carry2

            lax.fori_loop(0, _CH // 16, dist_group, 0)

        def phase(cc, S, Sp):
            @pl.when(cc < _NCH)
            def _():
                @pl.when(cc > 0)
                def _():
                    drain_wb(cc - 1, Sp)

                @pl.when(cc + 1 < _NCH)
                def _():
                    load_idx(cc + 1, Sp)
                    start_gathers(Sp)

                drain_gathers(S)
                compute_d2(S)
                add_rows(S)
                start_wb(cc, S)

        load_idx(0, 0)
        start_gathers(0)

        def body(j, carry):
            phase(2 * j, 0, 1)
            phase(2 * j + 1, 1, 0)
            return carry

        lax.fori_loop(0, (_NCH + 2) // 2, body, 0)
        drain_wb(_NCH - 1, (_NCH - 1) % 2)

    return k(pa, pb, x0, x1, x2, row, col)


def _edge_mlp_body(sa_ref, d2_ref, wd_ref, w2_ref, b2_ref, m_ref):
    dist = jnp.sqrt(d2_ref[...][:, 0:1])
    z = (sa_ref[...] + dist * wd_ref[...]).astype(_bf16)
    m1 = _silu(z)
    z2 = ((jnp.dot(m1, w2_ref[...], preferred_element_type=_f32)
           + b2_ref[...]).astype(_bf16))
    m_ref[...] = _silu(z2).astype(_f32)


def _edge_mlp(sa, d2, wd, w2, b2):
    full = lambda shp: pl.BlockSpec(shp, lambda i: (0,) * len(shp))
    return pl.pallas_call(
        _edge_mlp_body,
        grid=(_E // _EB,),
        in_specs=[
            pl.BlockSpec((_EB, _D), lambda i: (i, 0)),
            pl.BlockSpec((_EB, _CW), lambda i: (i, 0)),
            full((1, _D)), full((_D, _D)), full((1, _D)),
        ],
        out_specs=pl.BlockSpec((_EB, _D), lambda i: (i, 0)),
        out_shape=jax.ShapeDtypeStruct((_E, _D), _f32),
    )(sa, d2, wd, w2.astype(_bf16), b2)


def _scatter_sc(m, col, z128, zcnt):
    mesh = plsc.VectorSubcoreMesh(core_axis_name="c", subcore_axis_name="s",
                                  num_cores=_NC, num_subcores=_NS)

    @functools.partial(
        pl.kernel,
        out_type=(jax.ShapeDtypeStruct((_NC, _N, _D), _f32),
                  jax.ShapeDtypeStruct((_NW * _NP,), _f32)),
        mesh=mesh,
        compiler_params=pltpu.CompilerParams(needs_layout_passes=False),
        scratch_types=[
            pltpu.VMEM((2, _CH), _i32),
            pltpu.VMEM((_CH, _D), _f32),
            pltpu.VMEM((_CH, _D), _f32),
            pltpu.VMEM((_NP + 16,), _f32),
            pltpu.VMEM_SHARED((_N, _D), _f32),
            pltpu.SemaphoreType.DMA,
            pltpu.SemaphoreType.DMA,
            pltpu.SemaphoreType.DMA,
        ],
    )
    def k(m_h, col_h, z128_h, zcnt_h, msum_h, cnt_h, idx2, data0, data1,
          cntv, msh, seml, semsc0, semsc1):
        semsc = (semsc0, semsc1)
        data = (data0, data1)
        c = lax.axis_index("c")
        s = lax.axis_index("s")
        wid = c * _NS + s
        ebase = wid * _EPW
        pltpu.sync_copy(zcnt_h, cntv)
        lanes = lax.iota(_i32, 16)

        def count_chunk(S):
            def count16(g, carry2):
                ivvec = idx2[S, pl.ds(g * 16, 16)]
                for jj in range(16):
                    iv = ivvec[jj]
                    cbase = lax.shift_left(lax.shift_right_logical(iv, 3), 3)
                    lane = iv - cbase
                    cntv[pl.ds(cbase, 16)] = (cntv[pl.ds(cbase, 16)]
                                              + (lanes == lane).astype(_f32))
                return carry2

            lax.fori_loop(0, _CH // 16, count16, 0)

        def start_loads(cc, S):
            base = ebase + cc * _CH
            pltpu.async_copy(col_h.at[pl.ds(base, _CH)], idx2.at[S], seml)
            pltpu.async_copy(m_h.at[pl.ds(base, _CH)], data[S], seml)

        def drain_loads(cc, S):
            base = ebase + cc * _CH
            pltpu.make_async_copy(col_h.at[pl.ds(base, _CH)], idx2.at[S], seml).wait()
            pltpu.make_async_copy(m_h.at[pl.ds(base, _CH)], data[S], seml).wait()

        def start_scatter(S):
            pltpu.async_copy(data[S], msh.at[idx2.at[S]], semsc[S], add=True)

        def drain_scatter(S):
            pltpu.make_async_copy(data[S], msh.at[idx2.at[S]], semsc[S]).wait()

        @pl.when(s == 0)
        def _():
            pltpu.sync_copy(z128_h, msh)

        plsc.subcore_barrier()
        start_loads(0, 0)

        def phase(cc, S, Sp):
            @pl.when(cc < _NCH)
            def _():
                drain_loads(cc, S)

                @pl.when(cc > 0)
                def _():
                    drain_scatter(Sp)

                @pl.when(cc + 1 < _NCH)
                def _():
                    start_loads(cc + 1, Sp)

                start_scatter(S)
                count_chunk(S)

        def body(j, carry):
            phase(2 * j, 0, 1)
            phase(2 * j + 1, 1, 0)
            return carry

        lax.fori_loop(0, (_NCH + 2) // 2, body, 0)
        drain_scatter((_NCH - 1) % 2)
        pltpu.sync_copy(cntv.at[pl.ds(0, _NP)], cnt_h.at[pl.ds(wid * _NP, _NP)])
        plsc.subcore_barrier()
        rpt0 = 624

        @pl.when(s < _NS - 1)
        def _():
            rb = s * rpt0
            pltpu.sync_copy(msh.at[pl.ds(rb, rpt0)], msum_h.at[c, pl.ds(rb, rpt0)])

        @pl.when(s == _NS - 1)
        def _():
            rb = (_NS - 1) * rpt0
            rpt1 = _N - rb
            pltpu.sync_copy(msh.at[pl.ds(rb, rpt1)], msum_h.at[c, pl.ds(rb, rpt1)])

    return k(m, col, z128, zcnt)


def _cnt_transpose_body(ct_ref, eye_ref, out_ref):
    s = jnp.sum(ct_ref[...], axis=0)
    pieces = [lax.dot_general(eye_ref[...], s[k:k + 1, :],
                              (((1,), (1,)), ((), ())),
                              preferred_element_type=_f32)
              for k in range(8)]
    out_ref[...] = jnp.concatenate(pieces, axis=0)


def _cnt_transpose(cnt3, eye):
    return pl.pallas_call(
        _cnt_transpose_body,
        grid=(_NP // 1024,),
        in_specs=[
            pl.BlockSpec((_NW, 8, 128), lambda i: (0, i, 0)),
            pl.BlockSpec((128, 128), lambda i: (0, 0)),
        ],
        out_specs=pl.BlockSpec((1024, 1), lambda i: (i, 0)),
        out_shape=jax.ShapeDtypeStruct((_NP, 1), _f32),
    )(cnt3, eye)


def _node_body(h_ref, hn_ref, ms_ref, ct_ref, wna_ref, wnb_ref, bn1_ref,
               wn2_ref, bn2_ref, wm1_ref, bm1_ref, wm2_ref, bm2_ref,
               g2_ref, b2t_ref, out_ref):
    ms = ms_ref[0] + ms_ref[1]
    maggr = ms * (1.0 / jnp.maximum(ct_ref[...], 1.0))
    hn = hn_ref[...]
    bf = lambda v: v.astype(_bf16)
    z = (jnp.dot(bf(hn), bf(wna_ref[...]), preferred_element_type=_f32)
         + jnp.dot(bf(maggr), bf(wnb_ref[...]), preferred_element_type=_f32)
         + bn1_ref[...])
    a = _silu(z)
    h_delta = (jnp.dot(bf(a), bf(wn2_ref[...]), preferred_element_type=_f32)
               + bn2_ref[...])
    h1 = h_ref[...] + hn + h_delta
    mu = jnp.mean(h1, axis=1, keepdims=True)
    ctr = h1 - mu
    var = jnp.mean(ctr * ctr, axis=1, keepdims=True)
    hn2 = ctr * lax.rsqrt(var + 1e-5) * g2_ref[...] + b2t_ref[...]
    z2 = (jnp.dot(bf(hn2), bf(wm1_ref[...]), preferred_element_type=_f32)
          + bm1_ref[...])
    a2 = _silu(z2)
    out_ref[...] = (h1 + jnp.dot(bf(a2), bf(wm2_ref[...]), preferred_element_type=_f32)
                    + bm2_ref[...])


def _node_mlp(h, hn, msum2, cnt2, wna, wnb, bn1, wn2, bn2, wm1, bm1, wm2, bm2, g2, bt2):
    full = lambda shp: pl.BlockSpec(shp, lambda i: (0,) * len(shp))
    return pl.pallas_call(
        _node_body,
        grid=(_N // _RB,),
        in_specs=[
            pl.BlockSpec((_RB, _D), lambda i: (i, 0)),
            pl.BlockSpec((_RB, _D), lambda i: (i, 0)),
            pl.BlockSpec((_NC, _RB, _D), lambda i: (0, i, 0)),
            pl.BlockSpec((_RB, 1), lambda i: (i, 0)),
            full((_D, _D)), full((_D, _D)), full((1, _D)),
            full((_D, _D)), full((1, _D)),
            full((_D, _D)), full((1, _D)),
            full((_D, _D)), full((1, _D)),
            full((1, _D)), full((1, _D)),
        ],
        out_specs=pl.BlockSpec((_RB, _D), lambda i: (i, 0)),
        out_shape=jax.ShapeDtypeStruct((_N, _D), _f32),
    )(h, hn, msum2, cnt2, wna, wnb, bn1, wn2, bn2, wm1, bm1, wm2, bm2, g2, bt2)


def kernel(x, h, edge_index, We1, be1, We2, be2, Wn1, bn1, Wn2, bn2,
           Wm1, bm1, Wm2, bm2, g1, bt1, g2, bt2):
    ei = edge_index.astype(_i32)
    row = ei[0]
    col = ei[1]
    x0 = x[:, 0]
    x1 = x[:, 1]
    x2 = x[:, 2]

    w1a = We1[:_D]
    w1b = We1[_D:2 * _D]
    wd = We1[2 * _D].reshape(1, _D)
    r1 = lambda v: v.reshape(1, _D)

    hn, pa, pb = _ln_tables(h, r1(g1), r1(bt1), w1a, w1b, r1(be1))
    sa, d2 = _gather_sc(pa, pb, x0, x1, x2, row, col)
    m = _edge_mlp(sa, d2, wd, We2, r1(be2))

    z128 = jnp.zeros((_N, _D), _f32)
    zcnt = jnp.zeros((_NP + 16,), _f32)
    msum2, cntf = _scatter_sc(m, col, z128, zcnt)
    eye128 = jnp.eye(128, dtype=_f32)
    cntcol = _cnt_transpose(cntf.reshape(_NW, _NP // 128, 128), eye128)[:_N]

    return _node_mlp(h, hn, msum2, cntcol, Wn1[:_D], Wn1[_D:], r1(bn1),
                     Wn2, r1(bn2), Wm1, r1(bm1), Wm2, r1(bm2), r1(g2), r1(bt2))


# speedup vs baseline: 1.1744x; 1.0211x over previous
import functools

import jax
import jax.numpy as jnp
from jax import lax
from jax.experimental import pallas as pl
from jax.experimental.pallas import tpu as pltpu
from jax.experimental.pallas import tpu_sc as plsc

_N = 10000
_E = 320000
_D = 128
_NC = 2
_NS = 16
_NW = _NC * _NS
_EPW = _E // _NW
_CH = 80
_NCH = _EPW // _CH
_RB = 1000
_EB = 8000
_CW = 8
_NP = 10240

_f32 = jnp.float32
_bf16 = jnp.bfloat16


def _silu(z):
    return 0.5 * z * (1.0 + jnp.tanh(0.5 * z))
_i32 = jnp.int32


def _ln_tables_body(h_ref, g_ref, b_ref, wa_ref, wb_ref, bb_ref,
                    hn_ref, pa_ref, pb_ref):
    hb = h_ref[...]
    mu = jnp.mean(hb, axis=1, keepdims=True)
    ctr = hb - mu
    var = jnp.mean(ctr * ctr, axis=1, keepdims=True)
    hn = ctr * lax.rsqrt(var + 1e-5) * g_ref[...] + b_ref[...]
    hn_ref[...] = hn
    pa_ref[...] = jnp.dot(hn, wa_ref[...], preferred_element_type=_f32)
    pb_ref[...] = jnp.dot(hn, wb_ref[...], preferred_element_type=_f32) + bb_ref[...]


def _ln_tables(h, g1, bt1, w1a, w1b, be1):
    full = lambda shp: pl.BlockSpec(shp, lambda i: (0,) * len(shp))
    return pl.pallas_call(
        _ln_tables_body,
        grid=(_N // _RB,),
        in_specs=[
            pl.BlockSpec((_RB, _D), lambda i: (i, 0)),
            full((1, _D)), full((1, _D)),
            full((_D, _D)), full((_D, _D)), full((1, _D)),
        ],
        out_specs=[pl.BlockSpec((_RB, _D), lambda i: (i, 0))] * 3,
        out_shape=[jax.ShapeDtypeStruct((_N, _D), _f32)] * 3,
    )(h, g1, bt1, w1a, w1b, be1)


def _gather_sc(pa, pb, x0, x1, x2, row, col):
    mesh = plsc.VectorSubcoreMesh(core_axis_name="c", subcore_axis_name="s",
                                  num_cores=_NC, num_subcores=_NS)

    @functools.partial(
        pl.kernel,
        out_type=(jax.ShapeDtypeStruct((_E, _D), _f32),
                  jax.ShapeDtypeStruct((_E, _CW), _f32)),
        mesh=mesh,
        compiler_params=pltpu.CompilerParams(needs_layout_passes=False),
        scratch_types=[
            pltpu.VMEM((2, _CH), _i32),
            pltpu.VMEM((2, _CH), _i32),
            pltpu.VMEM((_CH, _D), _f32),
            pltpu.VMEM((_CH, _D), _f32),
            pltpu.VMEM((_CH, _D), _f32),
            pltpu.VMEM((_CH, _D), _f32),
            pltpu.VMEM((2, _CH, _CW), _f32),
            pltpu.VMEM((_N,), _f32),
            pltpu.VMEM((_N,), _f32),
            pltpu.VMEM((_N,), _f32),
            pltpu.SemaphoreType.DMA,
            pltpu.SemaphoreType.DMA,
            pltpu.SemaphoreType.DMA,
        ],
    )
    def k(pa_h, pb_h, x0_h, x1_h, x2_h, row_h, col_h,
          sa_h, d2_h,
          idxr2, idxc2, bufa0, bufa1, bufb0, bufb1, d2b2, x0v, x1v, x2v,
          semg0, semg1, semw):
        semg = (semg0, semg1)
        c = lax.axis_index("c")
        s = lax.axis_index("s")
        ebase = (c * _NS + s) * _EPW
        pltpu.sync_copy(x0_h, x0v)
        pltpu.sync_copy(x1_h, x1v)
        pltpu.sync_copy(x2_h, x2v)
        lanes = lax.iota(_i32, 16)
        zeros16 = jnp.zeros((16,), _i32)
        bufa = (bufa0, bufa1)
        bufb = (bufb0, bufb1)

        def load_idx(cc, S):
            base = ebase + cc * _CH
            pltpu.sync_copy(row_h.at[pl.ds(base, _CH)], idxr2.at[S])
            pltpu.sync_copy(col_h.at[pl.ds(base, _CH)], idxc2.at[S])

        def start_gathers(S):
            pltpu.async_copy(pa_h.at[idxr2.at[S]], bufa[S], semg[S])
            pltpu.async_copy(pb_h.at[idxc2.at[S]], bufb[S], semg[S])

        def drain_gathers(S):
            pltpu.make_async_copy(pa_h.at[idxr2.at[S]], bufa[S], semg[S]).wait()
            pltpu.make_async_copy(pb_h.at[idxc2.at[S]], bufb[S], semg[S]).wait()

        def start_wb(cc, S):
            base = ebase + cc * _CH
            pltpu.async_copy(bufa[S], sa_h.at[pl.ds(base, _CH)], semw)
            pltpu.async_copy(d2b2.at[S], d2_h.at[pl.ds(base, _CH)], semw)

        def drain_wb(cc, S):
            base = ebase + cc * _CH
            pltpu.make_async_copy(bufa[S], sa_h.at[pl.ds(base, _CH)], semw).wait()
            pltpu.make_async_copy(d2b2.at[S], d2_h.at[pl.ds(base, _CH)], semw).wait()

        def add_rows(S):
            def row_add(r, carry2):
                for l in range(_D // 16):
                    sl = pl.ds(l * 16, 16)
                    bufa[S][r, sl] = bufa[S][r, sl] + bufb[S][r, sl]
                return carry2

            lax.fori_loop(0, _CH, row_add, 0)

        def compute_d2(S):
            def dist_group(g, carry2):
                ir = idxr2[S, pl.ds(g * 16, 16)]
                ic = idxc2[S, pl.ds(g * 16, 16)]
                dx = plsc.load_gather(x0v, [ir]) - plsc.load_gather(x0v, [ic])
                dy = plsc.load_gather(x1v, [ir]) - plsc.load_gather(x1v, [ic])
                dz = plsc.load_gather(x2v, [ir]) - plsc.load_gather(x2v, [ic])
                d2v = dx * dx + dy * dy + dz * dz
                plsc.store_scatter(d2b2.at[S], [g * 16 + lanes, zeros16], d2v)
                return carry2

            lax.fori_loop(0, _CH // 16, dist_group, 0)

        def phase(cc, S, Sp):
            @pl.when(cc < _NCH)
            def _():
                @pl.when(cc > 0)
                def _():
                    drain_wb(cc - 1, Sp)

                @pl.when(cc + 1 < _NCH)
                def _():
                    load_idx(cc + 1, Sp)
                    start_gathers(Sp)

                drain_gathers(S)
                compute_d2(S)
                add_rows(S)
                start_wb(cc, S)

        load_idx(0, 0)
        start_gathers(0)

        def body(j, carry):
            phase(2 * j, 0, 1)
            phase(2 * j + 1, 1, 0)
            return carry

        lax.fori_loop(0, (_NCH + 2) // 2, body, 0)
        drain_wb(_NCH - 1, (_NCH - 1) % 2)

    return k(pa, pb, x0, x1, x2, row, col)


def _edge_mlp_body(sa_ref, d2_ref, wd_ref, w2_ref, b2_ref, m_ref):
    dist = jnp.sqrt(d2_ref[...][:, 0:1])
    z = (sa_ref[...] + dist * wd_ref[...]).astype(_bf16)
    m1 = _silu(z)
    z2 = ((jnp.dot(m1, w2_ref[...], preferred_element_type=_f32)
           + b2_ref[...]).astype(_bf16))
    m_ref[...] = _silu(z2).astype(_f32)


def _edge_mlp(sa, d2, wd, w2, b2):
    full = lambda shp: pl.BlockSpec(shp, lambda i: (0,) * len(shp))
    return pl.pallas_call(
        _edge_mlp_body,
        grid=(_E // _EB,),
        in_specs=[
            pl.BlockSpec((_EB, _D), lambda i: (i, 0)),
            pl.BlockSpec((_EB, _CW), lambda i: (i, 0)),
            full((1, _D)), full((_D, _D)), full((1, _D)),
        ],
        out_specs=pl.BlockSpec((_EB, _D), lambda i: (i, 0)),
        out_shape=jax.ShapeDtypeStruct((_E, _D), _f32),
    )(sa, d2, wd, w2.astype(_bf16), b2)


def _scatter_sc(m, col, z128, zcnt):
    mesh = plsc.VectorSubcoreMesh(core_axis_name="c", subcore_axis_name="s",
                                  num_cores=_NC, num_subcores=_NS)

    @functools.partial(
        pl.kernel,
        out_type=(jax.ShapeDtypeStruct((_NC, _N, _D), _f32),
                  jax.ShapeDtypeStruct((_NW * _NP,), _f32)),
        mesh=mesh,
        compiler_params=pltpu.CompilerParams(needs_layout_passes=False),
        scratch_types=[
            pltpu.VMEM((2, _CH), _i32),
            pltpu.VMEM((_CH, _D), _f32),
            pltpu.VMEM((_CH, _D), _f32),
            pltpu.VMEM((_NP + 16,), _f32),
            pltpu.VMEM_SHARED((_N, _D), _f32),
            pltpu.SemaphoreType.DMA,
            pltpu.SemaphoreType.DMA,
            pltpu.SemaphoreType.DMA,
        ],
    )
    def k(m_h, col_h, z128_h, zcnt_h, msum_h, cnt_h, idx2, data0, data1,
          cntv, msh, seml, semsc0, semsc1):
        semsc = (semsc0, semsc1)
        data = (data0, data1)
        c = lax.axis_index("c")
        s = lax.axis_index("s")
        wid = c * _NS + s
        ebase = wid * _EPW
        pltpu.sync_copy(zcnt_h, cntv)
        lanes = lax.iota(_i32, 16)

        def count_chunk(S):
            def count16(g, carry2):
                ivvec = idx2[S, pl.ds(g * 16, 16)]
                for jj in range(16):
                    iv = ivvec[jj]
                    cbase = lax.shift_left(lax.shift_right_logical(iv, 3), 3)
                    lane = iv - cbase
                    cntv[pl.ds(cbase, 16)] = (cntv[pl.ds(cbase, 16)]
                                              + (lanes == lane).astype(_f32))
                return carry2

            lax.fori_loop(0, _CH // 16, count16, 0)

        def start_loads(cc, S):
            base = ebase + cc * _CH
            pltpu.async_copy(col_h.at[pl.ds(base, _CH)], idx2.at[S], seml)
            pltpu.async_copy(m_h.at[pl.ds(base, _CH)], data[S], seml)

        def drain_loads(cc, S):
            base = ebase + cc * _CH
            pltpu.make_async_copy(col_h.at[pl.ds(base, _CH)], idx2.at[S], seml).wait()
            pltpu.make_async_copy(m_h.at[pl.ds(base, _CH)], data[S], seml).wait()

        def start_scatter(S):
            pltpu.async_copy(data[S], msh.at[idx2.at[S]], semsc[S], add=True)

        def drain_scatter(S):
            pltpu.make_async_copy(data[S], msh.at[idx2.at[S]], semsc[S]).wait()

        @pl.when(s == 0)
        def _():
            pltpu.sync_copy(z128_h, msh)

        plsc.subcore_barrier()
        start_loads(0, 0)

        def phase(cc, S, Sp):
            @pl.when(cc < _NCH)
            def _():
                drain_loads(cc, S)

                @pl.when(cc > 0)
                def _():
                    drain_scatter(Sp)

                @pl.when(cc + 1 < _NCH)
                def _():
                    start_loads(cc + 1, Sp)

                start_scatter(S)
                count_chunk(S)

        def body(j, carry):
            phase(2 * j, 0, 1)
            phase(2 * j + 1, 1, 0)
            return carry

        lax.fori_loop(0, (_NCH + 2) // 2, body, 0)
        drain_scatter((_NCH - 1) % 2)
        pltpu.sync_copy(cntv.at[pl.ds(0, _NP)], cnt_h.at[pl.ds(wid * _NP, _NP)])
        plsc.subcore_barrier()
        rpt0 = 624

        @pl.when(s < _NS - 1)
        def _():
            rb = s * rpt0
            pltpu.sync_copy(msh.at[pl.ds(rb, rpt0)], msum_h.at[c, pl.ds(rb, rpt0)])

        @pl.when(s == _NS - 1)
        def _():
            rb = (_NS - 1) * rpt0
            rpt1 = _N - rb
            pltpu.sync_copy(msh.at[pl.ds(rb, rpt1)], msum_h.at[c, pl.ds(rb, rpt1)])

    return k(m, col, z128, zcnt)


def _cnt_transpose_body(ct_ref, eye_ref, out_ref):
    s = jnp.sum(ct_ref[...], axis=0)
    pieces = [lax.dot_general(eye_ref[...], s[k:k + 1, :],
                              (((1,), (1,)), ((), ())),
                              preferred_element_type=_f32)
              for k in range(8)]
    out_ref[...] = jnp.concatenate(pieces, axis=0)


def _cnt_transpose(cnt3, eye):
    return pl.pallas_call(
        _cnt_transpose_body,
        grid=(_NP // 1024,),
        in_specs=[
            pl.BlockSpec((_NW, 8, 128), lambda i: (0, i, 0)),
            pl.BlockSpec((128, 128), lambda i: (0, 0)),
        ],
        out_specs=pl.BlockSpec((1024, 1), lambda i: (i, 0)),
        out_shape=jax.ShapeDtypeStruct((_NP, 1), _f32),
    )(cnt3, eye)


def _node_body(h_ref, hn_ref, ms_ref, ct_ref, wna_ref, wnb_ref, bn1_ref,
               wn2_ref, bn2_ref, wm1_ref, bm1_ref, wm2_ref, bm2_ref,
               g2_ref, b2t_ref, out_ref):
    ms = ms_ref[0] + ms_ref[1]
    maggr = ms * (1.0 / jnp.maximum(ct_ref[...], 1.0))
    hn = hn_ref[...]
    bf = lambda v: v.astype(_bf16)
    z = (jnp.dot(bf(hn), bf(wna_ref[...]), preferred_element_type=_f32)
         + jnp.dot(bf(maggr), bf(wnb_ref[...]), preferred_element_type=_f32)
         + bn1_ref[...])
    a = _silu(z)
    h_delta = (jnp.dot(bf(a), bf(wn2_ref[...]), preferred_element_type=_f32)
               + bn2_ref[...])
    h1 = h_ref[...] + hn + h_delta
    mu = jnp.mean(h1, axis=1, keepdims=True)
    ctr = h1 - mu
    var = jnp.mean(ctr * ctr, axis=1, keepdims=True)
    hn2 = ctr * lax.rsqrt(var + 1e-5) * g2_ref[...] + b2t_ref[...]
    z2 = (jnp.dot(bf(hn2), bf(wm1_ref[...]), preferred_element_type=_f32)
          + bm1_ref[...])
    a2 = _silu(z2)
    out_ref[...] = (h1 + jnp.dot(bf(a2), bf(wm2_ref[...]), preferred_element_type=_f32)
                    + bm2_ref[...])


def _node_mlp(h, hn, msum2, cnt2, wna, wnb, bn1, wn2, bn2, wm1, bm1, wm2, bm2, g2, bt2):
    full = lambda shp: pl.BlockSpec(shp, lambda i: (0,) * len(shp))
    return pl.pallas_call(
        _node_body,
        grid=(_N // _RB,),
        in_specs=[
            pl.BlockSpec((_RB, _D), lambda i: (i, 0)),
            pl.BlockSpec((_RB, _D), lambda i: (i, 0)),
            pl.BlockSpec((_NC, _RB, _D), lambda i: (0, i, 0)),
            pl.BlockSpec((_RB, 1), lambda i: (i, 0)),
            full((_D, _D)), full((_D, _D)), full((1, _D)),
            full((_D, _D)), full((1, _D)),
            full((_D, _D)), full((1, _D)),
            full((_D, _D)), full((1, _D)),
            full((1, _D)), full((1, _D)),
        ],
        out_specs=pl.BlockSpec((_RB, _D), lambda i: (i, 0)),
        out_shape=jax.ShapeDtypeStruct((_N, _D), _f32),
    )(h, hn, msum2, cnt2, wna, wnb, bn1, wn2, bn2, wm1, bm1, wm2, bm2, g2, bt2)


def kernel(x, h, edge_index, We1, be1, We2, be2, Wn1, bn1, Wn2, bn2,
           Wm1, bm1, Wm2, bm2, g1, bt1, g2, bt2):
    ei = edge_index.astype(_i32)
    row = ei[0]
    col = ei[1]
    x0 = x[:, 0]
    x1 = x[:, 1]
    x2 = x[:, 2]

    w1a = We1[:_D]
    w1b = We1[_D:2 * _D]
    wd = We1[2 * _D].reshape(1, _D)
    r1 = lambda v: v.reshape(1, _D)

    hn, pa, pb = _ln_tables(h, r1(g1), r1(bt1), w1a, w1b, r1(be1))
    sa, d2 = _gather_sc(pa, pb, x0, x1, x2, row, col)
    m = _edge_mlp(sa, d2, wd, We2, r1(be2))

    z128 = jnp.zeros((_N, _D), _f32)
    zcnt = jnp.zeros((_NP + 16,), _f32)
    msum2, cntf = _scatter_sc(m, col, z128, zcnt)
    eye128 = jnp.eye(128, dtype=_f32)
    cntcol = _cnt_transpose(cntf.reshape(_NW, _NP // 128, 128), eye128)[:_N]

    return _node_mlp(h, hn, msum2, cntcol, Wn1[:_D], Wn1[_D:], r1(bn1),
                     Wn2, r1(bn2), Wm1, r1(bm1), Wm2, r1(bm2), r1(g2), r1(bt2))


# speedup vs baseline: 1.1822x; 1.0066x over previous
import functools

import jax
import jax.numpy as jnp
from jax import lax
from jax.experimental import pallas as pl
from jax.experimental.pallas import tpu as pltpu
from jax.experimental.pallas import tpu_sc as plsc

_N = 10000
_E = 320000
_D = 128
_NC = 2
_NS = 16
_NW = _NC * _NS
_EPW = _E // _NW
_CH = 80
_NCH = _EPW // _CH
_RB = 2000
_EB = 16000
_CW = 8
_NP = 10240

_f32 = jnp.float32
_bf16 = jnp.bfloat16


def _silu(z):
    return 0.5 * z * (1.0 + jnp.tanh(0.5 * z))
_i32 = jnp.int32


def _ln_tables_body(h_ref, g_ref, b_ref, wa_ref, wb_ref, bb_ref,
                    hn_ref, pa_ref, pb_ref):
    hb = h_ref[...]
    mu = jnp.mean(hb, axis=1, keepdims=True)
    ctr = hb - mu
    var = jnp.mean(ctr * ctr, axis=1, keepdims=True)
    hn = ctr * lax.rsqrt(var + 1e-5) * g_ref[...] + b_ref[...]
    hn_ref[...] = hn
    pa_ref[...] = jnp.dot(hn, wa_ref[...], preferred_element_type=_f32)
    pb_ref[...] = jnp.dot(hn, wb_ref[...], preferred_element_type=_f32) + bb_ref[...]


def _ln_tables(h, g1, bt1, w1a, w1b, be1):
    full = lambda shp: pl.BlockSpec(shp, lambda i: (0,) * len(shp))
    return pl.pallas_call(
        _ln_tables_body,
        grid=(_N // _RB,),
        in_specs=[
            pl.BlockSpec((_RB, _D), lambda i: (i, 0)),
            full((1, _D)), full((1, _D)),
            full((_D, _D)), full((_D, _D)), full((1, _D)),
        ],
        out_specs=[pl.BlockSpec((_RB, _D), lambda i: (i, 0))] * 3,
        out_shape=[jax.ShapeDtypeStruct((_N, _D), _f32)] * 3,
    )(h, g1, bt1, w1a, w1b, be1)


def _gather_sc(pa, pb, x0, x1, x2, row, col):
    mesh = plsc.VectorSubcoreMesh(core_axis_name="c", subcore_axis_name="s",
                                  num_cores=_NC, num_subcores=_NS)

    @functools.partial(
        pl.kernel,
        out_type=(jax.ShapeDtypeStruct((_E, _D), _f32),
                  jax.ShapeDtypeStruct((_E, _CW), _f32)),
        mesh=mesh,
        compiler_params=pltpu.CompilerParams(needs_layout_passes=False),
        scratch_types=[
            pltpu.VMEM((2, _CH), _i32),
            pltpu.VMEM((2, _CH), _i32),
            pltpu.VMEM((_CH, _D), _f32),
            pltpu.VMEM((_CH, _D), _f32),
            pltpu.VMEM((_CH, _D), _f32),
            pltpu.VMEM((_CH, _D), _f32),
            pltpu.VMEM((2, _CH, _CW), _f32),
            pltpu.VMEM((_N,), _f32),
            pltpu.VMEM((_N,), _f32),
            pltpu.VMEM((_N,), _f32),
            pltpu.SemaphoreType.DMA,
            pltpu.SemaphoreType.DMA,
            pltpu.SemaphoreType.DMA,
        ],
    )
    def k(pa_h, pb_h, x0_h, x1_h, x2_h, row_h, col_h,
          sa_h, d2_h,
          idxr2, idxc2, bufa0, bufa1, bufb0, bufb1, d2b2, x0v, x1v, x2v,
          semg0, semg1, semw):
        semg = (semg0, semg1)
        c = lax.axis_index("c")
        s = lax.axis_index("s")
        ebase = (c * _NS + s) * _EPW
        pltpu.sync_copy(x0_h, x0v)
        pltpu.sync_copy(x1_h, x1v)
        pltpu.sync_copy(x2_h, x2v)
        lanes = lax.iota(_i32, 16)
        zeros16 = jnp.zeros((16,), _i32)
        bufa = (bufa0, bufa1)
        bufb = (bufb0, bufb1)

        def load_idx(cc, S):
            base = ebase + cc * _CH
            pltpu.sync_copy(row_h.at[pl.ds(base, _CH)], idxr2.at[S])
            pltpu.sync_copy(col_h.at[pl.ds(base, _CH)], idxc2.at[S])

        def start_gathers(S):
            pltpu.async_copy(pa_h.at[idxr2.at[S]], bufa[S], semg[S])
            pltpu.async_copy(pb_h.at[idxc2.at[S]], bufb[S], semg[S])

        def drain_gathers(S):
            pltpu.make_async_copy(pa_h.at[idxr2.at[S]], bufa[S], semg[S]).wait()
            pltpu.make_async_copy(pb_h.at[idxc2.at[S]], bufb[S], semg[S]).wait()

        def start_wb(cc, S):
            base = ebase + cc * _CH
            pltpu.async_copy(bufa[S], sa_h.at[pl.ds(base, _CH)], semw)
            pltpu.async_copy(d2b2.at[S], d2_h.at[pl.ds(base, _CH)], semw)

        def drain_wb(cc, S):
            base = ebase + cc * _CH
            pltpu.make_async_copy(bufa[S], sa_h.at[pl.ds(base, _CH)], semw).wait()
            pltpu.make_async_copy(d2b2.at[S], d2_h.at[pl.ds(base, _CH)], semw).wait()

        def add_rows(S):
            def row_add(r, carry2):
                for l in range(_D // 16):
                    sl = pl.ds(l * 16, 16)
                    bufa[S][r, sl] = bufa[S][r, sl] + bufb[S][r, sl]
                return carry2

            lax.fori_loop(0, _CH, row_add, 0)

        def compute_d2(S):
            def dist_group(g, carry2):
                ir = idxr2[S, pl.ds(g * 16, 16)]
                ic = idxc2[S, pl.ds(g * 16, 16)]
                dx = plsc.load_gather(x0v, [ir]) - plsc.load_gather(x0v, [ic])
                dy = plsc.load_gather(x1v, [ir]) - plsc.load_gather(x1v, [ic])
                dz = plsc.load_gather(x2v, [ir]) - plsc.load_gather(x2v, [ic])
                d2v = dx * dx + dy * dy + dz * dz
                plsc.store_scatter(d2b2.at[S], [g * 16 + lanes, zeros16], d2v)
                return carry2

            lax.fori_loop(0, _CH // 16, dist_group, 0)

        def phase(cc, S, Sp):
            @pl.when(cc < _NCH)
            def _():
                @pl.when(cc > 0)
                def _():
                    drain_wb(cc - 1, Sp)

                @pl.when(cc + 1 < _NCH)
                def _():
                    load_idx(cc + 1, Sp)
                    start_gathers(Sp)

                drain_gathers(S)
                compute_d2(S)
                add_rows(S)
                start_wb(cc, S)

        load_idx(0, 0)
        start_gathers(0)

        def body(j, carry):
            phase(2 * j, 0, 1)
            phase(2 * j + 1, 1, 0)
            return carry

        lax.fori_loop(0, (_NCH + 2) // 2, body, 0)
        drain_wb(_NCH - 1, (_NCH - 1) % 2)

    return k(pa, pb, x0, x1, x2, row, col)


def _edge_mlp_body(sa_ref, d2_ref, wd_ref, w2_ref, b2_ref, m_ref):
    dist = jnp.sqrt(d2_ref[...][:, 0:1])
    z = (sa_ref[...] + dist * wd_ref[...]).astype(_bf16)
    m1 = _silu(z)
    z2 = ((jnp.dot(m1, w2_ref[...], preferred_element_type=_f32)
           + b2_ref[...]).astype(_bf16))
    m_ref[...] = _silu(z2).astype(_f32)


def _edge_mlp(sa, d2, wd, w2, b2):
    full = lambda shp: pl.BlockSpec(shp, lambda i: (0,) * len(shp))
    return pl.pallas_call(
        _edge_mlp_body,
        grid=(_E // _EB,),
        in_specs=[
            pl.BlockSpec((_EB, _D), lambda i: (i, 0)),
            pl.BlockSpec((_EB, _CW), lambda i: (i, 0)),
            full((1, _D)), full((_D, _D)), full((1, _D)),
        ],
        out_specs=pl.BlockSpec((_EB, _D), lambda i: (i, 0)),
        out_shape=jax.ShapeDtypeStruct((_E, _D), _f32),
    )(sa, d2, wd, w2.astype(_bf16), b2)


def _scatter_sc(m, col, z128, zcnt):
    mesh = plsc.VectorSubcoreMesh(core_axis_name="c", subcore_axis_name="s",
                                  num_cores=_NC, num_subcores=_NS)

    @functools.partial(
        pl.kernel,
        out_type=(jax.ShapeDtypeStruct((_NC, _N, _D), _f32),
                  jax.ShapeDtypeStruct((_NW * _NP,), _f32)),
        mesh=mesh,
        compiler_params=pltpu.CompilerParams(needs_layout_passes=False),
        scratch_types=[
            pltpu.VMEM((2, _CH), _i32),
            pltpu.VMEM((_CH, _D), _f32),
            pltpu.VMEM((_CH, _D), _f32),
            pltpu.VMEM((_NP + 16,), _f32),
            pltpu.VMEM_SHARED((_N, _D), _f32),
            pltpu.SemaphoreType.DMA,
            pltpu.SemaphoreType.DMA,
            pltpu.SemaphoreType.DMA,
        ],
    )
    def k(m_h, col_h, z128_h, zcnt_h, msum_h, cnt_h, idx2, data0, data1,
          cntv, msh, seml, semsc0, semsc1):
        semsc = (semsc0, semsc1)
        data = (data0, data1)
        c = lax.axis_index("c")
        s = lax.axis_index("s")
        wid = c * _NS + s
        ebase = wid * _EPW
        pltpu.sync_copy(zcnt_h, cntv)
        lanes = lax.iota(_i32, 16)

        def count_chunk(S):
            def count16(g, carry2):
                ivvec = idx2[S, pl.ds(g * 16, 16)]
                for jj in range(16):
                    iv = ivvec[jj]
                    cbase = lax.shift_left(lax.shift_right_logical(iv, 3), 3)
                    lane = iv - cbase
                    cntv[pl.ds(cbase, 16)] = (cntv[pl.ds(cbase, 16)]
                                              + (lanes == lane).astype(_f32))
                return carry2

            lax.fori_loop(0, _CH // 16, count16, 0)

        def start_loads(cc, S):
            base = ebase + cc * _CH
            pltpu.async_copy(col_h.at[pl.ds(base, _CH)], idx2.at[S], seml)
            pltpu.async_copy(m_h.at[pl.ds(base, _CH)], data[S], seml)

        def drain_loads(cc, S):
            base = ebase + cc * _CH
            pltpu.make_async_copy(col_h.at[pl.ds(base, _CH)], idx2.at[S], seml).wait()
            pltpu.make_async_copy(m_h.at[pl.ds(base, _CH)], data[S], seml).wait()

        def start_scatter(S):
            pltpu.async_copy(data[S], msh.at[idx2.at[S]], semsc[S], add=True)

        def drain_scatter(S):
            pltpu.make_async_copy(data[S], msh.at[idx2.at[S]], semsc[S]).wait()

        @pl.when(s == 0)
        def _():
            pltpu.sync_copy(z128_h, msh)

        plsc.subcore_barrier()
        start_loads(0, 0)

        def phase(cc, S, Sp):
            @pl.when(cc < _NCH)
            def _():
                drain_loads(cc, S)

                @pl.when(cc > 0)
                def _():
                    drain_scatter(Sp)

                @pl.when(cc + 1 < _NCH)
                def _():
                    start_loads(cc + 1, Sp)

                start_scatter(S)
                count_chunk(S)

        def body(j, carry):
            phase(2 * j, 0, 1)
            phase(2 * j + 1, 1, 0)
            return carry

        lax.fori_loop(0, (_NCH + 2) // 2, body, 0)
        drain_scatter((_NCH - 1) % 2)
        pltpu.sync_copy(cntv.at[pl.ds(0, _NP)], cnt_h.at[pl.ds(wid * _NP, _NP)])
        plsc.subcore_barrier()
        rpt0 = 624

        @pl.when(s < _NS - 1)
        def _():
            rb = s * rpt0
            pltpu.sync_copy(msh.at[pl.ds(rb, rpt0)], msum_h.at[c, pl.ds(rb, rpt0)])

        @pl.when(s == _NS - 1)
        def _():
            rb = (_NS - 1) * rpt0
            rpt1 = _N - rb
            pltpu.sync_copy(msh.at[pl.ds(rb, rpt1)], msum_h.at[c, pl.ds(rb, rpt1)])

    return k(m, col, z128, zcnt)


def _cnt_transpose_body(ct_ref, eye_ref, out_ref):
    s = jnp.sum(ct_ref[...], axis=0)
    pieces = [lax.dot_general(eye_ref[...], s[k:k + 1, :],
                              (((1,), (1,)), ((), ())),
                              preferred_element_type=_f32)
              for k in range(8)]
    out_ref[...] = jnp.concatenate(pieces, axis=0)


def _cnt_transpose(cnt3, eye):
    return pl.pallas_call(
        _cnt_transpose_body,
        grid=(_NP // 1024,),
        in_specs=[
            pl.BlockSpec((_NW, 8, 128), lambda i: (0, i, 0)),
            pl.BlockSpec((128, 128), lambda i: (0, 0)),
        ],
        out_specs=pl.BlockSpec((1024, 1), lambda i: (i, 0)),
        out_shape=jax.ShapeDtypeStruct((_NP, 1), _f32),
    )(cnt3, eye)


def _node_body(h_ref, hn_ref, ms_ref, ct_ref, wna_ref, wnb_ref, bn1_ref,
               wn2_ref, bn2_ref, wm1_ref, bm1_ref, wm2_ref, bm2_ref,
               g2_ref, b2t_ref, out_ref):
    ms = ms_ref[0] + ms_ref[1]
    maggr = ms * (1.0 / jnp.maximum(ct_ref[...], 1.0))
    hn = hn_ref[...]
    bf = lambda v: v.astype(_bf16)
    z = (jnp.dot(bf(hn), bf(wna_ref[...]), preferred_element_type=_f32)
         + jnp.dot(bf(maggr), bf(wnb_ref[...]), preferred_element_type=_f32)
         + bn1_ref[...])
    a = _silu(z)
    h_delta = (jnp.dot(bf(a), bf(wn2_ref[...]), preferred_element_type=_f32)
               + bn2_ref[...])
    h1 = h_ref[...] + hn + h_delta
    mu = jnp.mean(h1, axis=1, keepdims=True)
    ctr = h1 - mu
    var = jnp.mean(ctr * ctr, axis=1, keepdims=True)
    hn2 = ctr * lax.rsqrt(var + 1e-5) * g2_ref[...] + b2t_ref[...]
    z2 = (jnp.dot(bf(hn2), bf(wm1_ref[...]), preferred_element_type=_f32)
          + bm1_ref[...])
    a2 = _silu(z2)
    out_ref[...] = (h1 + jnp.dot(bf(a2), bf(wm2_ref[...]), preferred_element_type=_f32)
                    + bm2_ref[...])


def _node_mlp(h, hn, msum2, cnt2, wna, wnb, bn1, wn2, bn2, wm1, bm1, wm2, bm2, g2, bt2):
    full = lambda shp: pl.BlockSpec(shp, lambda i: (0,) * len(shp))
    return pl.pallas_call(
        _node_body,
        grid=(_N // _RB,),
        in_specs=[
            pl.BlockSpec((_RB, _D), lambda i: (i, 0)),
            pl.BlockSpec((_RB, _D), lambda i: (i, 0)),
            pl.BlockSpec((_NC, _RB, _D), lambda i: (0, i, 0)),
            pl.BlockSpec((_RB, 1), lambda i: (i, 0)),
            full((_D, _D)), full((_D, _D)), full((1, _D)),
            full((_D, _D)), full((1, _D)),
            full((_D, _D)), full((1, _D)),
            full((_D, _D)), full((1, _D)),
            full((1, _D)), full((1, _D)),
        ],
        out_specs=pl.BlockSpec((_RB, _D), lambda i: (i, 0)),
        out_shape=jax.ShapeDtypeStruct((_N, _D), _f32),
    )(h, hn, msum2, cnt2, wna, wnb, bn1, wn2, bn2, wm1, bm1, wm2, bm2, g2, bt2)


def kernel(x, h, edge_index, We1, be1, We2, be2, Wn1, bn1, Wn2, bn2,
           Wm1, bm1, Wm2, bm2, g1, bt1, g2, bt2):
    ei = edge_index.astype(_i32)
    row = ei[0]
    col = ei[1]
    x0 = x[:, 0]
    x1 = x[:, 1]
    x2 = x[:, 2]

    w1a = We1[:_D]
    w1b = We1[_D:2 * _D]
    wd = We1[2 * _D].reshape(1, _D)
    r1 = lambda v: v.reshape(1, _D)

    hn, pa, pb = _ln_tables(h, r1(g1), r1(bt1), w1a, w1b, r1(be1))
    sa, d2 = _gather_sc(pa, pb, x0, x1, x2, row, col)
    m = _edge_mlp(sa, d2, wd, We2, r1(be2))

    z128 = jnp.zeros((_N, _D), _f32)
    zcnt = jnp.zeros((_NP + 16,), _f32)
    msum2, cntf = _scatter_sc(m, col, z128, zcnt)
    eye128 = jnp.eye(128, dtype=_f32)
    cntcol = _cnt_transpose(cntf.reshape(_NW, _NP // 128, 128), eye128)[:_N]

    return _node_mlp(h, hn, msum2, cntcol, Wn1[:_D], Wn1[_D:], r1(bn1),
                     Wn2, r1(bn2), Wm1, r1(bm1), Wm2, r1(bm2), r1(g2), r1(bt2))
